# Initial kernel scaffold; baseline (speedup 1.0000x reference)
import jax
import jax.numpy as jnp
from jax.experimental import pallas as pl


def kernel(x_paper, x_author, edge_index_cites, edge_index_writes, edge_index_rev, batch, Wl1c, bl1c, Wr1c, Wl1w, bl1w, Wr1w, Wl1r, bl1r, Wr1r, Wl2c, bl2c, Wr2c, Wl2w, bl2w, Wr2w, Wl2r, bl2r, Wr2r, Wlin, blin):
    raise NotImplementedError("write your pallas kernel here")


# speedup vs baseline: 1.1947x; 1.1947x over previous
import functools

import jax
import jax.numpy as jnp
from jax import lax
from jax.experimental import pallas as pl
from jax.experimental.pallas import tpu as pltpu
from jax.experimental.pallas import tpu_sc as plsc

N = 50000
E = 200000
D = 128
G = 64
C = 32

NC = 2
NS = 16
RNG = 12544
NP = 4 * RNG
ACC = RNG + 256
EPT = 12544
EP = NS * EPT
NB = EPT // 128
RB = 256
GB = NP // RB


def _agg_body(nrels, with_counts, *refs):
    pos = 0
    tabs = refs[pos:pos + nrels]; pos += nrels
    edges = refs[pos:pos + 2 * nrels]; pos += 2 * nrels
    aggs = refs[pos:pos + nrels]; pos += nrels
    cnts = refs[pos:pos + nrels] if with_counts else ()
    if with_counts:
        pos += nrels
    (acc_sp, cnt_sp, srcv, dstv, gsm, ssm, rowbuf, onesv,
     zcnt, cntv, sem) = refs[pos:]

    core = lax.axis_index("c")
    sub = lax.axis_index("s")
    zero16 = jnp.zeros((16,), jnp.float32)
    one16 = jnp.ones((16,), jnp.float32)
    iota16 = lax.iota(jnp.int32, 16)

    @pl.loop(0, 50)
    def _(k):
        zcnt[pl.ds(k * 16, 16)] = zero16

    @pl.loop(0, 4)
    def _(k):
        onesv[pl.ds(k * 16, 16)] = one16

    trash_s = RNG + ((sub * 16 + iota16) & 255)
    trash_g = sub * 16 + iota16

    for r in range(nrels):
        tab = tabs[r]
        src_h, dst_h = edges[2 * r], edges[2 * r + 1]
        for p in range(2):
            rid = 2 * p + core
            lo = rid * RNG

            @pl.loop(0, 512)
            def _(k):
                rowbuf[k >> 3, pl.ds((k & 7) * 16, 16)] = zero16

            zbase = sub * (ACC // NS)
            for k in range(12):
                pltpu.sync_copy(rowbuf, acc_sp.at[pl.ds(zbase + 64 * k, 64)])
            pltpu.sync_copy(rowbuf.at[pl.ds(0, 32)],
                            acc_sp.at[pl.ds(zbase + 768, 32)])
            if with_counts:
                pltpu.sync_copy(zcnt, cnt_sp.at[pl.ds(zbase, 800)])
            plsc.subcore_barrier()

            @pl.loop(0, 7)
            def _(c):
                pltpu.sync_copy(src_h.at[sub, pl.ds(c * 112, 112)], srcv)
                pltpu.sync_copy(dst_h.at[sub, pl.ds(c * 112, 112)], dstv)

                @pl.loop(0, 28)
                def _(q):
                    for jj in range(4):
                        j = q * 4 + jj
                        s16 = srcv[j]
                        d16 = dstv[j]
                        m = (d16 >= lo) & (d16 < lo + RNG)
                        gsm[0, pl.ds(jj * 16, 16)] = jnp.where(
                            m, s16, trash_g)
                        ssm[0, pl.ds(jj * 16, 16)] = jnp.where(
                            m, d16 - lo, trash_s)
                    pltpu.async_copy(tab.at[gsm.at[0]], rowbuf, sem).wait()
                    pltpu.sync_copy(rowbuf, acc_sp.at[ssm.at[0]], add=True)
                    if with_counts:
                        pltpu.sync_copy(onesv, cnt_sp.at[ssm.at[0]],
                                        add=True)

            plsc.subcore_barrier()
            wbase = sub * (RNG // NS)
            pltpu.sync_copy(acc_sp.at[pl.ds(wbase, RNG // NS)],
                            aggs[r].at[pl.ds(lo + wbase, RNG // NS)])
            if with_counts:
                pltpu.sync_copy(
                    cnt_sp.at[pl.ds(pl.multiple_of(wbase, 8), RNG // NS)],
                    cntv)
                pltpu.sync_copy(cntv, cnts[r].at[rid * NS + sub])
            plsc.subcore_barrier()


def _make_agg(nrels, with_counts):
    outs = [jax.ShapeDtypeStruct((NP, D), jnp.float32) for _ in range(nrels)]
    if with_counts:
        outs += [jax.ShapeDtypeStruct((64, RNG // NS), jnp.float32)
                 for _ in range(nrels)]
    mesh = plsc.VectorSubcoreMesh(core_axis_name="c", subcore_axis_name="s",
                                  num_cores=NC, num_subcores=NS)
    return pl.kernel(
        functools.partial(_agg_body, nrels, with_counts),
        out_type=tuple(outs),
        mesh=mesh,
        scratch_types=[
            pltpu.VMEM_SHARED((ACC, D), jnp.float32),
            pltpu.VMEM_SHARED((ACC,), jnp.float32),
            pltpu.VMEM((112, 16), jnp.int32),
            pltpu.VMEM((112, 16), jnp.int32),
            pltpu.VMEM((1, 64), jnp.int32),
            pltpu.VMEM((1, 64), jnp.int32),
            pltpu.VMEM((64, D), jnp.float32),
            pltpu.VMEM((64,), jnp.float32),
            pltpu.VMEM((800,), jnp.float32),
            pltpu.VMEM((RNG // NS,), jnp.float32),
            pltpu.SemaphoreType.DMA,
        ],
        compiler_params=pltpu.CompilerParams(use_tc_tiling_on_sc=False),
        name=f"sc_agg{nrels}",
    )


_agg3 = _make_agg(3, True)
_agg2 = _make_agg(2, False)


def _layer1_body(aggc, aggw, aggr, cc, cw, cr, xp, xa,
                 wlc, wlw, wlr, wrp, wra, bp, ba, xp1, xa1):
    invc = 1.0 / jnp.maximum(cc[...], 1.0)
    invw = 1.0 / jnp.maximum(cw[...], 1.0)
    invr = 1.0 / jnp.maximum(cr[...], 1.0)
    f32 = jnp.float32
    hp = (jnp.dot(aggc[...] * invc, wlc[...], preferred_element_type=f32)
          + jnp.dot(aggw[...] * invw, wlw[...], preferred_element_type=f32)
          + jnp.dot(xp[...], wrp[...], preferred_element_type=f32) + bp[...])
    ha = (jnp.dot(aggr[...] * invr, wlr[...], preferred_element_type=f32)
          + jnp.dot(xa[...], wra[...], preferred_element_type=f32) + ba[...])
    xp1[...] = jnp.maximum(hp, 0.0)
    xa1[...] = jnp.maximum(ha, 0.0)


def _layer2_body(i, aggc, aggw, cc, cw, xp1, bat, wlc, wlw, wrp, bp,
                 wlin, blin, pooled, cntb, final):
    invc = 1.0 / jnp.maximum(cc[...], 1.0)
    invw = 1.0 / jnp.maximum(cw[...], 1.0)
    f32 = jnp.float32
    hp = (jnp.dot(aggc[...] * invc, wlc[...], preferred_element_type=f32)
          + jnp.dot(aggw[...] * invw, wlw[...], preferred_element_type=f32)
          + jnp.dot(xp1[...], wrp[...], preferred_element_type=f32) + bp[...])
    xp2 = jnp.maximum(hp, 0.0)
    bb = bat[0]
    oh = (lax.broadcasted_iota(jnp.int32, (G, RB), 0)
          == jnp.broadcast_to(bb, (G, RB))).astype(f32)

    @pl.when(i == 0)
    def _():
        pooled[...] = jnp.zeros((G, D), f32)
        cntb[...] = jnp.zeros((G, D), f32)

    pooled[...] += jnp.dot(oh, xp2, preferred_element_type=f32)
    cntb[...] += jnp.broadcast_to(jnp.sum(oh, axis=1, keepdims=True), (G, D))

    @pl.when(i == GB - 1)
    def _():
        inv = 1.0 / jnp.maximum(cntb[...], 1.0)
        final[...] = (jnp.dot(pooled[...] * inv, wlin[...],
                              preferred_element_type=f32) + blin[...])


def _l2_with_i(*args):
    _layer2_body(pl.program_id(0), *args)


def _l1_with_i(*args):
    _layer1_body(*args)


_row = pl.BlockSpec((RB, D), lambda i: (i, 0))
_col1 = pl.BlockSpec((RB, 1), lambda i: (i, 0))
_wfull = pl.BlockSpec((D, D), lambda i: (0, 0))
_bfull = pl.BlockSpec((1, D), lambda i: (0, 0))

_layer1 = pl.pallas_call(
    _l1_with_i,
    grid=(GB,),
    in_specs=[_row, _row, _row, _col1, _col1, _col1, _row, _row,
              _wfull, _wfull, _wfull, _wfull, _wfull, _bfull, _bfull],
    out_specs=[_row, _row],
    out_shape=[jax.ShapeDtypeStruct((NP, D), jnp.float32),
               jax.ShapeDtypeStruct((NP, D), jnp.float32)],
    compiler_params=pltpu.CompilerParams(
        dimension_semantics=("arbitrary",)),
)

_layer2 = pl.pallas_call(
    _l2_with_i,
    grid=(GB,),
    in_specs=[_row, _row, _col1, _col1, _row,
              pl.BlockSpec((1, 1, RB), lambda i: (i, 0, 0)),
              _wfull, _wfull, _wfull, _bfull,
              pl.BlockSpec((D, C), lambda i: (0, 0)),
              pl.BlockSpec((1, C), lambda i: (0, 0))],
    out_specs=[pl.BlockSpec((G, D), lambda i: (0, 0)),
               pl.BlockSpec((G, D), lambda i: (0, 0)),
               pl.BlockSpec((G, C), lambda i: (0, 0))],
    out_shape=[jax.ShapeDtypeStruct((G, D), jnp.float32),
               jax.ShapeDtypeStruct((G, D), jnp.float32),
               jax.ShapeDtypeStruct((G, C), jnp.float32)],
    compiler_params=pltpu.CompilerParams(
        dimension_semantics=("arbitrary",)),
)


def _pad_edges(ei):
    src = jnp.concatenate([ei[0], jnp.zeros((EP - E,), jnp.int32)])
    dst = jnp.concatenate([ei[1], jnp.full((EP - E,), 1 << 28, jnp.int32)])
    return src.reshape(NS, EPT // 16, 16), dst.reshape(NS, EPT // 16, 16)


def kernel(x_paper, x_author, edge_index_cites, edge_index_writes,
           edge_index_rev, batch, Wl1c, bl1c, Wr1c, Wl1w, bl1w, Wr1w,
           Wl1r, bl1r, Wr1r, Wl2c, bl2c, Wr2c, Wl2w, bl2w, Wr2w,
           Wl2r, bl2r, Wr2r, Wlin, blin):
    padn = jnp.zeros((NP - N, D), jnp.float32)
    xp = jnp.concatenate([x_paper, padn])
    xa = jnp.concatenate([x_author, padn])
    sc_, dc_ = _pad_edges(edge_index_cites)
    sw_, dw_ = _pad_edges(edge_index_writes)
    sr_, dr_ = _pad_edges(edge_index_rev)
    bat = jnp.concatenate([batch, jnp.full((NP - N,), G, jnp.int32)])
    bat = bat.reshape(GB, 1, RB)

    aggc, aggw, aggr, cc, cw, cr = _agg3(xp, xa, xp,
                                         sc_, dc_, sw_, dw_, sr_, dr_)
    cc = cc.reshape(NP, 1)
    cw = cw.reshape(NP, 1)
    cr = cr.reshape(NP, 1)

    xp1, xa1 = _layer1(aggc, aggw, aggr, cc, cw, cr, xp, xa,
                       Wl1c, Wl1w, Wl1r, (Wr1c + Wr1w), Wr1r,
                       (bl1c + bl1w).reshape(1, D), bl1r.reshape(1, D))

    aggc2, aggw2 = _agg2(xp1, xa1, sc_, dc_, sw_, dw_)

    _, _, final = _layer2(aggc2, aggw2, cc, cw, xp1, bat,
                          Wl2c, Wl2w, (Wr2c + Wr2w),
                          (bl2c + bl2w).reshape(1, D),
                          Wlin, blin.reshape(1, C))
    return final


# speedup vs baseline: 2.6004x; 2.1766x over previous
import functools

import jax
import jax.numpy as jnp
from jax import lax
from jax.experimental import pallas as pl
from jax.experimental.pallas import tpu as pltpu
from jax.experimental.pallas import tpu_sc as plsc

N = 50000
E = 200000
D = 128
G = 64
C = 32

NC = 2
NS = 16
NPASS = 4
RNG = 6272
NP = 8 * RNG
ACC = RNG + 256
EPT = 12544
EP = NS * EPT
CAP = 12800
RB = 256
GB = NP // RB


def _agg_body(nrels, with_counts, *refs):
    pos = 0
    tabs = refs[pos:pos + nrels]; pos += nrels
    edges = refs[pos:pos + 2 * nrels]; pos += 2 * nrels
    aggs = refs[pos:pos + nrels]; pos += nrels
    cnts = refs[pos:pos + nrels] if with_counts else ()
    if with_counts:
        pos += nrels
    (acc_sp, cnt_sp, srcv, dstv, cg, cs, rowbuf, onesv,
     zcnt, cntv, sem) = refs[pos:]

    core = lax.axis_index("c")
    sub = lax.axis_index("s")
    zero16 = jnp.zeros((16,), jnp.float32)
    one16 = jnp.ones((16,), jnp.float32)
    iota16 = lax.iota(jnp.int32, 16)

    @pl.loop(0, ACC // NS // 16 + 1)
    def _(k):
        zcnt[pl.ds(k * 16, 16)] = zero16

    @pl.loop(0, 8)
    def _(k):
        onesv[pl.ds(k * 16, 16)] = one16

    trash_s = RNG + ((sub * 16 + iota16) & 255)
    trash_g = sub * 16 + iota16

    for r in range(nrels):
        tab = tabs[r]
        src_h, dst_h = edges[2 * r], edges[2 * r + 1]
        for p in range(NPASS):
            rid = 2 * p + core
            lo = rid * RNG

            @pl.loop(0, 1024)
            def _(k):
                rowbuf[k >> 3, pl.ds((k & 7) * 16, 16)] = zero16

            zbase = sub * (ACC // NS)
            for k in range(3):
                pltpu.sync_copy(rowbuf,
                                acc_sp.at[pl.ds(zbase + 128 * k, 128)])
            pltpu.sync_copy(rowbuf.at[pl.ds(0, ACC // NS - 384)],
                            acc_sp.at[pl.ds(zbase + 384, ACC // NS - 384)])
            if with_counts:
                pltpu.sync_copy(zcnt.at[pl.ds(0, ACC // NS)],
                                cnt_sp.at[pl.ds(zbase, ACC // NS)])

            @pl.loop(0, CAP // 16)
            def _(k):
                cg[pl.ds(k * 16, 16)] = trash_g
                cs[pl.ds(k * 16, 16)] = trash_s

            plsc.subcore_barrier()

            def _chunk(c, cnt):
                pltpu.sync_copy(src_h.at[sub, pl.ds(c * 112, 112)], srcv)
                pltpu.sync_copy(dst_h.at[sub, pl.ds(c * 112, 112)], dstv)

                def _scan(j, cnt):
                    s16 = srcv[j]
                    d16 = dstv[j]
                    m = (d16 >= lo) & (d16 < lo + RNG)
                    cum = plsc.cumsum(m.astype(jnp.int32))
                    pos = cnt + cum - 1
                    plsc.store_scatter(cg, [pos], s16, mask=m)
                    plsc.store_scatter(cs, [pos], d16 - lo, mask=m)
                    return cnt + jnp.sum(m.astype(jnp.int32))

                return lax.fori_loop(0, 112, _scan, cnt)

            cnt = lax.fori_loop(0, 7, _chunk, jnp.int32(0))
            nb = (cnt + 127) >> 7

            def _batch(b, _):
                gslice = cg.at[pl.ds(b * 128, 128)]
                sslice = cs.at[pl.ds(b * 128, 128)]
                pltpu.async_copy(tab.at[gslice], rowbuf, sem).wait()
                pltpu.sync_copy(rowbuf, acc_sp.at[sslice], add=True)
                if with_counts:
                    pltpu.sync_copy(onesv, cnt_sp.at[sslice], add=True)
                return 0

            lax.fori_loop(0, nb, _batch, 0)

            plsc.subcore_barrier()
            wbase = sub * (RNG // NS)
            pltpu.sync_copy(acc_sp.at[pl.ds(wbase, RNG // NS)],
                            aggs[r].at[pl.ds(lo + wbase, RNG // NS)])
            if with_counts:
                pltpu.sync_copy(
                    cnt_sp.at[pl.ds(pl.multiple_of(wbase, 8), RNG // NS)],
                    cntv)
                pltpu.sync_copy(cntv, cnts[r].at[rid * NS + sub])
            plsc.subcore_barrier()


def _make_agg(nrels, with_counts):
    outs = [jax.ShapeDtypeStruct((NP, D), jnp.float32) for _ in range(nrels)]
    if with_counts:
        outs += [jax.ShapeDtypeStruct((2 * NPASS * NS, RNG // NS),
                                      jnp.float32)
                 for _ in range(nrels)]
    mesh = plsc.VectorSubcoreMesh(core_axis_name="c", subcore_axis_name="s",
                                  num_cores=NC, num_subcores=NS)
    return pl.kernel(
        functools.partial(_agg_body, nrels, with_counts),
        out_type=tuple(outs),
        mesh=mesh,
        scratch_types=[
            pltpu.VMEM_SHARED((ACC, D), jnp.float32),
            pltpu.VMEM_SHARED((ACC,), jnp.float32),
            pltpu.VMEM((112, 16), jnp.int32),
            pltpu.VMEM((112, 16), jnp.int32),
            pltpu.VMEM((CAP,), jnp.int32),
            pltpu.VMEM((CAP,), jnp.int32),
            pltpu.VMEM((128, D), jnp.float32),
            pltpu.VMEM((128,), jnp.float32),
            pltpu.VMEM((800,), jnp.float32),
            pltpu.VMEM((RNG // NS,), jnp.float32),
            pltpu.SemaphoreType.DMA,
        ],
        compiler_params=pltpu.CompilerParams(use_tc_tiling_on_sc=False,
                                             needs_layout_passes=False),
        name=f"sc_agg{nrels}",
    )


_agg3 = _make_agg(3, True)
_agg2 = _make_agg(2, False)


def _layer1_body(aggc, aggw, aggr, cc, cw, cr, xp, xa,
                 wlc, wlw, wlr, wrp, wra, bp, ba, xp1, xa1):
    invc = 1.0 / jnp.maximum(cc[...], 1.0)
    invw = 1.0 / jnp.maximum(cw[...], 1.0)
    invr = 1.0 / jnp.maximum(cr[...], 1.0)
    f32 = jnp.float32
    hp = (jnp.dot(aggc[...] * invc, wlc[...], preferred_element_type=f32)
          + jnp.dot(aggw[...] * invw, wlw[...], preferred_element_type=f32)
          + jnp.dot(xp[...], wrp[...], preferred_element_type=f32) + bp[...])
    ha = (jnp.dot(aggr[...] * invr, wlr[...], preferred_element_type=f32)
          + jnp.dot(xa[...], wra[...], preferred_element_type=f32) + ba[...])
    xp1[...] = jnp.maximum(hp, 0.0)
    xa1[...] = jnp.maximum(ha, 0.0)


def _layer2_body(i, aggc, aggw, cc, cw, xp1, bat, wlc, wlw, wrp, bp,
                 wlin, blin, pooled, cntb, final):
    invc = 1.0 / jnp.maximum(cc[...], 1.0)
    invw = 1.0 / jnp.maximum(cw[...], 1.0)
    f32 = jnp.float32
    hp = (jnp.dot(aggc[...] * invc, wlc[...], preferred_element_type=f32)
          + jnp.dot(aggw[...] * invw, wlw[...], preferred_element_type=f32)
          + jnp.dot(xp1[...], wrp[...], preferred_element_type=f32) + bp[...])
    xp2 = jnp.maximum(hp, 0.0)
    bb = bat[0]
    oh = (lax.broadcasted_iota(jnp.int32, (G, RB), 0)
          == jnp.broadcast_to(bb, (G, RB))).astype(f32)

    @pl.when(i == 0)
    def _():
        pooled[...] = jnp.zeros((G, D), f32)
        cntb[...] = jnp.zeros((G, D), f32)

    pooled[...] += jnp.dot(oh, xp2, preferred_element_type=f32)
    cntb[...] += jnp.broadcast_to(jnp.sum(oh, axis=1, keepdims=True), (G, D))

    @pl.when(i == GB - 1)
    def _():
        inv = 1.0 / jnp.maximum(cntb[...], 1.0)
        final[...] = (jnp.dot(pooled[...] * inv, wlin[...],
                              preferred_element_type=f32) + blin[...])


def _l2_with_i(*args):
    _layer2_body(pl.program_id(0), *args)


def _l1_with_i(*args):
    _layer1_body(*args)


_row = pl.BlockSpec((RB, D), lambda i: (i, 0))
_col1 = pl.BlockSpec((RB, 1), lambda i: (i, 0))
_wfull = pl.BlockSpec((D, D), lambda i: (0, 0))
_bfull = pl.BlockSpec((1, D), lambda i: (0, 0))

_layer1 = pl.pallas_call(
    _l1_with_i,
    grid=(GB,),
    in_specs=[_row, _row, _row, _col1, _col1, _col1, _row, _row,
              _wfull, _wfull, _wfull, _wfull, _wfull, _bfull, _bfull],
    out_specs=[_row, _row],
    out_shape=[jax.ShapeDtypeStruct((NP, D), jnp.float32),
               jax.ShapeDtypeStruct((NP, D), jnp.float32)],
    compiler_params=pltpu.CompilerParams(
        dimension_semantics=("arbitrary",)),
)

_layer2 = pl.pallas_call(
    _l2_with_i,
    grid=(GB,),
    in_specs=[_row, _row, _col1, _col1, _row,
              pl.BlockSpec((1, 1, RB), lambda i: (i, 0, 0)),
              _wfull, _wfull, _wfull, _bfull,
              pl.BlockSpec((D, C), lambda i: (0, 0)),
              pl.BlockSpec((1, C), lambda i: (0, 0))],
    out_specs=[pl.BlockSpec((G, D), lambda i: (0, 0)),
               pl.BlockSpec((G, D), lambda i: (0, 0)),
               pl.BlockSpec((G, C), lambda i: (0, 0))],
    out_shape=[jax.ShapeDtypeStruct((G, D), jnp.float32),
               jax.ShapeDtypeStruct((G, D), jnp.float32),
               jax.ShapeDtypeStruct((G, C), jnp.float32)],
    compiler_params=pltpu.CompilerParams(
        dimension_semantics=("arbitrary",)),
)


def _pad_edges(ei):
    src = jnp.concatenate([ei[0], jnp.zeros((EP - E,), jnp.int32)])
    dst = jnp.concatenate([ei[1], jnp.full((EP - E,), 1 << 28, jnp.int32)])
    return src.reshape(NS, EPT // 16, 16), dst.reshape(NS, EPT // 16, 16)


def kernel(x_paper, x_author, edge_index_cites, edge_index_writes,
           edge_index_rev, batch, Wl1c, bl1c, Wr1c, Wl1w, bl1w, Wr1w,
           Wl1r, bl1r, Wr1r, Wl2c, bl2c, Wr2c, Wl2w, bl2w, Wr2w,
           Wl2r, bl2r, Wr2r, Wlin, blin):
    padn = jnp.zeros((NP - N, D), jnp.float32)
    xp = jnp.concatenate([x_paper, padn])
    xa = jnp.concatenate([x_author, padn])
    sc_, dc_ = _pad_edges(edge_index_cites)
    sw_, dw_ = _pad_edges(edge_index_writes)
    sr_, dr_ = _pad_edges(edge_index_rev)
    bat = jnp.concatenate([batch, jnp.full((NP - N,), G, jnp.int32)])
    bat = bat.reshape(GB, 1, RB)

    aggc, aggw, aggr, cc, cw, cr = _agg3(xp, xa, xp,
                                         sc_, dc_, sw_, dw_, sr_, dr_)
    cc = cc.reshape(NP, 1)
    cw = cw.reshape(NP, 1)
    cr = cr.reshape(NP, 1)

    xp1, xa1 = _layer1(aggc, aggw, aggr, cc, cw, cr, xp, xa,
                       Wl1c, Wl1w, Wl1r, (Wr1c + Wr1w), Wr1r,
                       (bl1c + bl1w).reshape(1, D), bl1r.reshape(1, D))

    aggc2, aggw2 = _agg2(xp1, xa1, sc_, dc_, sw_, dw_)

    _, _, final = _layer2(aggc2, aggw2, cc, cw, xp1, bat,
                          Wl2c, Wl2w, (Wr2c + Wr2w),
                          (bl2c + bl2w).reshape(1, D),
                          Wlin, blin.reshape(1, C))
    return final


# speedup vs baseline: 2.7509x; 1.0579x over previous
import functools

import jax
import jax.numpy as jnp
from jax import lax
from jax.experimental import pallas as pl
from jax.experimental.pallas import tpu as pltpu
from jax.experimental.pallas import tpu_sc as plsc

N = 50000
E = 200000
D = 128
G = 64
C = 32

NC = 2
NS = 16
NPASS = 4
RNG = 6272
NP = 8 * RNG
ACC = RNG + 256
EPT = 12544
EP = NS * EPT
CAP = 12800
RB = 256
GB = NP // RB


def _agg_body(nrels, with_counts, *refs):
    pos = 0
    tabs = refs[pos:pos + nrels]; pos += nrels
    edges = refs[pos:pos + 2 * nrels]; pos += 2 * nrels
    aggs = refs[pos:pos + nrels]; pos += nrels
    cnts = refs[pos:pos + nrels] if with_counts else ()
    if with_counts:
        pos += nrels
    (acc_sp, cnt_sp, srcv, dstv, cg, cs, rowbuf0, rowbuf1, onesv,
     zcnt, cntv, gsem0, gsem1, ssem0, ssem1) = refs[pos:]
    rowbufs = (rowbuf0, rowbuf1)
    gsems = (gsem0, gsem1)
    ssems = (ssem0, ssem1)
    rowbuf = rowbuf0

    core = lax.axis_index("c")
    sub = lax.axis_index("s")
    zero16 = jnp.zeros((16,), jnp.float32)
    one16 = jnp.ones((16,), jnp.float32)
    iota16 = lax.iota(jnp.int32, 16)

    @pl.loop(0, ACC // NS // 16 + 1)
    def _(k):
        zcnt[pl.ds(k * 16, 16)] = zero16

    @pl.loop(0, 8)
    def _(k):
        onesv[pl.ds(k * 16, 16)] = one16

    trash_s = RNG + ((sub * 16 + iota16) & 255)
    trash_g = sub * 16 + iota16

    for r in range(nrels):
        tab = tabs[r]
        src_h, dst_h = edges[2 * r], edges[2 * r + 1]
        for p in range(NPASS):
            rid = 2 * p + core
            lo = rid * RNG

            @pl.loop(0, 1024)
            def _(k):
                rowbuf[k >> 3, pl.ds((k & 7) * 16, 16)] = zero16

            zbase = sub * (ACC // NS)
            for k in range(3):
                pltpu.sync_copy(rowbuf,
                                acc_sp.at[pl.ds(zbase + 128 * k, 128)])
            pltpu.sync_copy(rowbuf.at[pl.ds(0, ACC // NS - 384)],
                            acc_sp.at[pl.ds(zbase + 384, ACC // NS - 384)])
            if with_counts:
                pltpu.sync_copy(zcnt.at[pl.ds(0, ACC // NS)],
                                cnt_sp.at[pl.ds(zbase, ACC // NS)])

            @pl.loop(0, CAP // 16)
            def _(k):
                cg[pl.ds(k * 16, 16)] = trash_g
                cs[pl.ds(k * 16, 16)] = trash_s

            plsc.subcore_barrier()

            def _chunk(c, cnt):
                pltpu.sync_copy(src_h.at[sub, pl.ds(c * 112, 112)], srcv)
                pltpu.sync_copy(dst_h.at[sub, pl.ds(c * 112, 112)], dstv)

                def _scan(j, cnt):
                    s16 = srcv[j]
                    d16 = dstv[j]
                    m = (d16 >= lo) & (d16 < lo + RNG)
                    cum = plsc.cumsum(m.astype(jnp.int32))
                    pos = cnt + cum - 1
                    plsc.store_scatter(cg, [pos], s16, mask=m)
                    plsc.store_scatter(cs, [pos], d16 - lo, mask=m)
                    return cnt + jnp.sum(m.astype(jnp.int32))

                return lax.fori_loop(0, 112, _scan, cnt)

            cnt = lax.fori_loop(0, 7, _chunk, jnp.int32(0))
            nb = (cnt + 127) >> 7

            def _drain(u):
                dummy = cs.at[pl.ds(0, 128)]
                pltpu.make_async_copy(rowbufs[u], acc_sp.at[dummy],
                                      ssems[u]).wait()
                if with_counts:
                    pltpu.make_async_copy(onesv, cnt_sp.at[dummy],
                                          ssems[u]).wait()

            def _half(h, _):
                for u in range(2):
                    b = h * 2 + u

                    @pl.when(b < nb)
                    def _():
                        @pl.when(b >= 2)
                        def _():
                            _drain(u)
                        gslice = cg.at[pl.ds(b * 128, 128)]
                        pltpu.async_copy(tab.at[gslice], rowbufs[u],
                                         gsems[u])
                for u in range(2):
                    b = h * 2 + u

                    @pl.when(b < nb)
                    def _():
                        gslice = cg.at[pl.ds(b * 128, 128)]
                        sslice = cs.at[pl.ds(b * 128, 128)]
                        pltpu.make_async_copy(tab.at[gslice], rowbufs[u],
                                              gsems[u]).wait()
                        pltpu.async_copy(rowbufs[u], acc_sp.at[sslice],
                                         ssems[u], add=True)
                        if with_counts:
                            pltpu.async_copy(onesv, cnt_sp.at[sslice],
                                             ssems[u], add=True)
                return 0

            lax.fori_loop(0, (nb + 1) >> 1, _half, 0)
            for u in range(2):
                @pl.when(nb > u)
                def _():
                    _drain(u)

            plsc.subcore_barrier()
            wbase = sub * (RNG // NS)
            pltpu.sync_copy(acc_sp.at[pl.ds(wbase, RNG // NS)],
                            aggs[r].at[pl.ds(lo + wbase, RNG // NS)])
            if with_counts:
                pltpu.sync_copy(
                    cnt_sp.at[pl.ds(pl.multiple_of(wbase, 8), RNG // NS)],
                    cntv)
                pltpu.sync_copy(cntv, cnts[r].at[rid * NS + sub])
            plsc.subcore_barrier()


def _make_agg(nrels, with_counts):
    outs = [jax.ShapeDtypeStruct((NP, D), jnp.float32) for _ in range(nrels)]
    if with_counts:
        outs += [jax.ShapeDtypeStruct((2 * NPASS * NS, RNG // NS),
                                      jnp.float32)
                 for _ in range(nrels)]
    mesh = plsc.VectorSubcoreMesh(core_axis_name="c", subcore_axis_name="s",
                                  num_cores=NC, num_subcores=NS)
    return pl.kernel(
        functools.partial(_agg_body, nrels, with_counts),
        out_type=tuple(outs),
        mesh=mesh,
        scratch_types=[
            pltpu.VMEM_SHARED((ACC, D), jnp.float32),
            pltpu.VMEM_SHARED((ACC,), jnp.float32),
            pltpu.VMEM((112, 16), jnp.int32),
            pltpu.VMEM((112, 16), jnp.int32),
            pltpu.VMEM((CAP,), jnp.int32),
            pltpu.VMEM((CAP,), jnp.int32),
            pltpu.VMEM((128, D), jnp.float32),
            pltpu.VMEM((128, D), jnp.float32),
            pltpu.VMEM((128,), jnp.float32),
            pltpu.VMEM((800,), jnp.float32),
            pltpu.VMEM((RNG // NS,), jnp.float32),
            pltpu.SemaphoreType.DMA,
            pltpu.SemaphoreType.DMA,
            pltpu.SemaphoreType.DMA,
            pltpu.SemaphoreType.DMA,
        ],
        compiler_params=pltpu.CompilerParams(use_tc_tiling_on_sc=False,
                                             needs_layout_passes=False),
        name=f"sc_agg{nrels}",
    )


_agg3 = _make_agg(3, True)
_agg2 = _make_agg(2, False)


def _layer1_body(aggc, aggw, aggr, cc, cw, cr, xp, xa,
                 wlc, wlw, wlr, wrp, wra, bp, ba, xp1, xa1):
    invc = 1.0 / jnp.maximum(cc[...], 1.0)
    invw = 1.0 / jnp.maximum(cw[...], 1.0)
    invr = 1.0 / jnp.maximum(cr[...], 1.0)
    f32 = jnp.float32
    hp = (jnp.dot(aggc[...] * invc, wlc[...], preferred_element_type=f32)
          + jnp.dot(aggw[...] * invw, wlw[...], preferred_element_type=f32)
          + jnp.dot(xp[...], wrp[...], preferred_element_type=f32) + bp[...])
    ha = (jnp.dot(aggr[...] * invr, wlr[...], preferred_element_type=f32)
          + jnp.dot(xa[...], wra[...], preferred_element_type=f32) + ba[...])
    xp1[...] = jnp.maximum(hp, 0.0)
    xa1[...] = jnp.maximum(ha, 0.0)


def _layer2_body(i, aggc, aggw, cc, cw, xp1, bat, wlc, wlw, wrp, bp,
                 wlin, blin, pooled, cntb, final):
    invc = 1.0 / jnp.maximum(cc[...], 1.0)
    invw = 1.0 / jnp.maximum(cw[...], 1.0)
    f32 = jnp.float32
    hp = (jnp.dot(aggc[...] * invc, wlc[...], preferred_element_type=f32)
          + jnp.dot(aggw[...] * invw, wlw[...], preferred_element_type=f32)
          + jnp.dot(xp1[...], wrp[...], preferred_element_type=f32) + bp[...])
    xp2 = jnp.maximum(hp, 0.0)
    bb = bat[0]
    oh = (lax.broadcasted_iota(jnp.int32, (G, RB), 0)
          == jnp.broadcast_to(bb, (G, RB))).astype(f32)

    @pl.when(i == 0)
    def _():
        pooled[...] = jnp.zeros((G, D), f32)
        cntb[...] = jnp.zeros((G, D), f32)

    pooled[...] += jnp.dot(oh, xp2, preferred_element_type=f32)
    cntb[...] += jnp.broadcast_to(jnp.sum(oh, axis=1, keepdims=True), (G, D))

    @pl.when(i == GB - 1)
    def _():
        inv = 1.0 / jnp.maximum(cntb[...], 1.0)
        final[...] = (jnp.dot(pooled[...] * inv, wlin[...],
                              preferred_element_type=f32) + blin[...])


def _l2_with_i(*args):
    _layer2_body(pl.program_id(0), *args)


def _l1_with_i(*args):
    _layer1_body(*args)


_row = pl.BlockSpec((RB, D), lambda i: (i, 0))
_col1 = pl.BlockSpec((RB, 1), lambda i: (i, 0))
_wfull = pl.BlockSpec((D, D), lambda i: (0, 0))
_bfull = pl.BlockSpec((1, D), lambda i: (0, 0))

_layer1 = pl.pallas_call(
    _l1_with_i,
    grid=(GB,),
    in_specs=[_row, _row, _row, _col1, _col1, _col1, _row, _row,
              _wfull, _wfull, _wfull, _wfull, _wfull, _bfull, _bfull],
    out_specs=[_row, _row],
    out_shape=[jax.ShapeDtypeStruct((NP, D), jnp.float32),
               jax.ShapeDtypeStruct((NP, D), jnp.float32)],
    compiler_params=pltpu.CompilerParams(
        dimension_semantics=("arbitrary",)),
)

_layer2 = pl.pallas_call(
    _l2_with_i,
    grid=(GB,),
    in_specs=[_row, _row, _col1, _col1, _row,
              pl.BlockSpec((1, 1, RB), lambda i: (i, 0, 0)),
              _wfull, _wfull, _wfull, _bfull,
              pl.BlockSpec((D, C), lambda i: (0, 0)),
              pl.BlockSpec((1, C), lambda i: (0, 0))],
    out_specs=[pl.BlockSpec((G, D), lambda i: (0, 0)),
               pl.BlockSpec((G, D), lambda i: (0, 0)),
               pl.BlockSpec((G, C), lambda i: (0, 0))],
    out_shape=[jax.ShapeDtypeStruct((G, D), jnp.float32),
               jax.ShapeDtypeStruct((G, D), jnp.float32),
               jax.ShapeDtypeStruct((G, C), jnp.float32)],
    compiler_params=pltpu.CompilerParams(
        dimension_semantics=("arbitrary",)),
)


def _pad_edges(ei):
    src = jnp.concatenate([ei[0], jnp.zeros((EP - E,), jnp.int32)])
    dst = jnp.concatenate([ei[1], jnp.full((EP - E,), 1 << 28, jnp.int32)])
    return src.reshape(NS, EPT // 16, 16), dst.reshape(NS, EPT // 16, 16)


def kernel(x_paper, x_author, edge_index_cites, edge_index_writes,
           edge_index_rev, batch, Wl1c, bl1c, Wr1c, Wl1w, bl1w, Wr1w,
           Wl1r, bl1r, Wr1r, Wl2c, bl2c, Wr2c, Wl2w, bl2w, Wr2w,
           Wl2r, bl2r, Wr2r, Wlin, blin):
    padn = jnp.zeros((NP - N, D), jnp.float32)
    xp = jnp.concatenate([x_paper, padn])
    xa = jnp.concatenate([x_author, padn])
    sc_, dc_ = _pad_edges(edge_index_cites)
    sw_, dw_ = _pad_edges(edge_index_writes)
    sr_, dr_ = _pad_edges(edge_index_rev)
    bat = jnp.concatenate([batch, jnp.full((NP - N,), G, jnp.int32)])
    bat = bat.reshape(GB, 1, RB)

    aggc, aggw, aggr, cc, cw, cr = _agg3(xp, xa, xp,
                                         sc_, dc_, sw_, dw_, sr_, dr_)
    cc = cc.reshape(NP, 1)
    cw = cw.reshape(NP, 1)
    cr = cr.reshape(NP, 1)

    xp1, xa1 = _layer1(aggc, aggw, aggr, cc, cw, cr, xp, xa,
                       Wl1c, Wl1w, Wl1r, (Wr1c + Wr1w), Wr1r,
                       (bl1c + bl1w).reshape(1, D), bl1r.reshape(1, D))

    aggc2, aggw2 = _agg2(xp1, xa1, sc_, dc_, sw_, dw_)

    _, _, final = _layer2(aggc2, aggw2, cc, cw, xp1, bat,
                          Wl2c, Wl2w, (Wr2c + Wr2w),
                          (bl2c + bl2w).reshape(1, D),
                          Wlin, blin.reshape(1, C))
    return final


# speedup vs baseline: 3.0664x; 1.1147x over previous
import functools

import jax
import jax.numpy as jnp
from jax import lax
from jax.experimental import pallas as pl
from jax.experimental.pallas import tpu as pltpu
from jax.experimental.pallas import tpu_sc as plsc

N = 50000
E = 200000
D = 128
G = 64
C = 32

NC = 2
NS = 16
NPASS = 4
RNG = 6272
NP = 8 * RNG
ACC = RNG + 256
EPT = 12544
EP = NS * EPT
CAP = 12800
RB = 256
GB = NP // RB


def _agg_body(nrels, with_counts, *refs):
    pos = 0
    tabs = refs[pos:pos + nrels]; pos += nrels
    edges = refs[pos:pos + 2 * nrels]; pos += 2 * nrels
    aggs = refs[pos:pos + nrels]; pos += nrels
    cnts = refs[pos:pos + nrels] if with_counts else ()
    if with_counts:
        pos += nrels
    (acc_sp, cnt_sp, srcv, dstv, cg, cs, rowbuf0, rowbuf1, onesv,
     zcnt, cntv, gsem0, gsem1, ssem0, ssem1) = refs[pos:]
    rowbufs = (rowbuf0, rowbuf1)
    gsems = (gsem0, gsem1)
    ssems = (ssem0, ssem1)
    rowbuf = rowbuf0

    core = lax.axis_index("c")
    sub = lax.axis_index("s")
    zero16 = jnp.zeros((16,), jnp.float32)
    one16 = jnp.ones((16,), jnp.float32)
    iota16 = lax.iota(jnp.int32, 16)

    @pl.loop(0, ACC // NS // 16 + 1)
    def _(k):
        zcnt[pl.ds(k * 16, 16)] = zero16

    @pl.loop(0, 8)
    def _(k):
        onesv[pl.ds(k * 16, 16)] = one16

    trash_s = RNG + ((sub * 16 + iota16) & 255)
    trash_g = sub * 16 + iota16

    for r in range(nrels):
        tab = tabs[r]
        src_h, dst_h = edges[2 * r], edges[2 * r + 1]
        for p in range(NPASS):
            rid = 2 * p + core
            lo = rid * RNG

            @pl.loop(0, 1024)
            def _(k):
                rowbuf[k >> 3, pl.ds((k & 7) * 16, 16)] = zero16

            zbase = sub * (ACC // NS)
            for k in range(3):
                pltpu.sync_copy(rowbuf,
                                acc_sp.at[pl.ds(zbase + 128 * k, 128)])
            pltpu.sync_copy(rowbuf.at[pl.ds(0, ACC // NS - 384)],
                            acc_sp.at[pl.ds(zbase + 384, ACC // NS - 384)])
            if with_counts:
                pltpu.sync_copy(zcnt.at[pl.ds(0, ACC // NS)],
                                cnt_sp.at[pl.ds(zbase, ACC // NS)])

            plsc.subcore_barrier()

            def _chunk(c, cnt):
                pltpu.sync_copy(src_h.at[sub, pl.ds(c * 112, 112)], srcv)
                pltpu.sync_copy(dst_h.at[sub, pl.ds(c * 112, 112)], dstv)

                def _scan(q, cnt):
                    ss, ds_, ms, cums = [], [], [], []
                    for jj in range(4):
                        j = q * 4 + jj
                        s16 = srcv[j]
                        d16 = dstv[j]
                        m = (d16 >= lo) & (d16 < lo + RNG)
                        ss.append(s16)
                        ds_.append(d16)
                        ms.append(m)
                        cums.append(plsc.cumsum(m.astype(jnp.int32)))
                    for jj in range(4):
                        pos = cnt + cums[jj] - 1
                        plsc.store_scatter(cg, [pos], ss[jj], mask=ms[jj])
                        plsc.store_scatter(cs, [pos], ds_[jj] - lo,
                                           mask=ms[jj])
                        cnt = cnt + cums[jj][15]
                    return cnt

                return lax.fori_loop(0, 28, _scan, cnt)

            cnt = lax.fori_loop(0, 7, _chunk, jnp.int32(0))

            for k in range(8):
                tpos = cnt + 16 * k + iota16
                plsc.store_scatter(cg, [tpos], trash_g)
                plsc.store_scatter(cs, [tpos], trash_s)

            nb = (cnt + 127) >> 7

            def _drain(u):
                dummy = cs.at[pl.ds(0, 128)]
                pltpu.make_async_copy(rowbufs[u], acc_sp.at[dummy],
                                      ssems[u]).wait()
                if with_counts:
                    pltpu.make_async_copy(onesv, cnt_sp.at[dummy],
                                          ssems[u]).wait()

            def _half(h, _):
                for u in range(2):
                    b = h * 2 + u

                    @pl.when(b < nb)
                    def _():
                        @pl.when(b >= 2)
                        def _():
                            _drain(u)
                        gslice = cg.at[pl.ds(b * 128, 128)]
                        pltpu.async_copy(tab.at[gslice], rowbufs[u],
                                         gsems[u])
                for u in range(2):
                    b = h * 2 + u

                    @pl.when(b < nb)
                    def _():
                        gslice = cg.at[pl.ds(b * 128, 128)]
                        sslice = cs.at[pl.ds(b * 128, 128)]
                        pltpu.make_async_copy(tab.at[gslice], rowbufs[u],
                                              gsems[u]).wait()
                        pltpu.async_copy(rowbufs[u], acc_sp.at[sslice],
                                         ssems[u], add=True)
                        if with_counts:
                            pltpu.async_copy(onesv, cnt_sp.at[sslice],
                                             ssems[u], add=True)
                return 0

            lax.fori_loop(0, (nb + 1) >> 1, _half, 0)
            for u in range(2):
                @pl.when(nb > u)
                def _():
                    _drain(u)

            plsc.subcore_barrier()
            wbase = sub * (RNG // NS)
            pltpu.sync_copy(acc_sp.at[pl.ds(wbase, RNG // NS)],
                            aggs[r].at[pl.ds(lo + wbase, RNG // NS)])
            if with_counts:
                pltpu.sync_copy(
                    cnt_sp.at[pl.ds(pl.multiple_of(wbase, 8), RNG // NS)],
                    cntv)
                pltpu.sync_copy(cntv, cnts[r].at[rid * NS + sub])
            plsc.subcore_barrier()


def _make_agg(nrels, with_counts):
    outs = [jax.ShapeDtypeStruct((NP, D), jnp.float32) for _ in range(nrels)]
    if with_counts:
        outs += [jax.ShapeDtypeStruct((2 * NPASS * NS, RNG // NS),
                                      jnp.float32)
                 for _ in range(nrels)]
    mesh = plsc.VectorSubcoreMesh(core_axis_name="c", subcore_axis_name="s",
                                  num_cores=NC, num_subcores=NS)
    return pl.kernel(
        functools.partial(_agg_body, nrels, with_counts),
        out_type=tuple(outs),
        mesh=mesh,
        scratch_types=[
            pltpu.VMEM_SHARED((ACC, D), jnp.float32),
            pltpu.VMEM_SHARED((ACC,), jnp.float32),
            pltpu.VMEM((112, 16), jnp.int32),
            pltpu.VMEM((112, 16), jnp.int32),
            pltpu.VMEM((CAP,), jnp.int32),
            pltpu.VMEM((CAP,), jnp.int32),
            pltpu.VMEM((128, D), jnp.float32),
            pltpu.VMEM((128, D), jnp.float32),
            pltpu.VMEM((128,), jnp.float32),
            pltpu.VMEM((800,), jnp.float32),
            pltpu.VMEM((RNG // NS,), jnp.float32),
            pltpu.SemaphoreType.DMA,
            pltpu.SemaphoreType.DMA,
            pltpu.SemaphoreType.DMA,
            pltpu.SemaphoreType.DMA,
        ],
        compiler_params=pltpu.CompilerParams(use_tc_tiling_on_sc=False,
                                             needs_layout_passes=False),
        name=f"sc_agg{nrels}",
    )


_agg3 = _make_agg(3, True)
_agg2 = _make_agg(2, False)


def _layer1_body(aggc, aggw, aggr, cc, cw, cr, xp, xa,
                 wlc, wlw, wlr, wrp, wra, bp, ba, xp1, xa1):
    invc = 1.0 / jnp.maximum(cc[...], 1.0)
    invw = 1.0 / jnp.maximum(cw[...], 1.0)
    invr = 1.0 / jnp.maximum(cr[...], 1.0)
    f32 = jnp.float32
    hp = (jnp.dot(aggc[...] * invc, wlc[...], preferred_element_type=f32)
          + jnp.dot(aggw[...] * invw, wlw[...], preferred_element_type=f32)
          + jnp.dot(xp[...], wrp[...], preferred_element_type=f32) + bp[...])
    ha = (jnp.dot(aggr[...] * invr, wlr[...], preferred_element_type=f32)
          + jnp.dot(xa[...], wra[...], preferred_element_type=f32) + ba[...])
    xp1[...] = jnp.maximum(hp, 0.0)
    xa1[...] = jnp.maximum(ha, 0.0)


def _layer2_body(i, aggc, aggw, cc, cw, xp1, bat, wlc, wlw, wrp, bp,
                 wlin, blin, pooled, cntb, final):
    invc = 1.0 / jnp.maximum(cc[...], 1.0)
    invw = 1.0 / jnp.maximum(cw[...], 1.0)
    f32 = jnp.float32
    hp = (jnp.dot(aggc[...] * invc, wlc[...], preferred_element_type=f32)
          + jnp.dot(aggw[...] * invw, wlw[...], preferred_element_type=f32)
          + jnp.dot(xp1[...], wrp[...], preferred_element_type=f32) + bp[...])
    xp2 = jnp.maximum(hp, 0.0)
    bb = bat[0]
    oh = (lax.broadcasted_iota(jnp.int32, (G, RB), 0)
          == jnp.broadcast_to(bb, (G, RB))).astype(f32)

    @pl.when(i == 0)
    def _():
        pooled[...] = jnp.zeros((G, D), f32)
        cntb[...] = jnp.zeros((G, D), f32)

    pooled[...] += jnp.dot(oh, xp2, preferred_element_type=f32)
    cntb[...] += jnp.broadcast_to(jnp.sum(oh, axis=1, keepdims=True), (G, D))

    @pl.when(i == GB - 1)
    def _():
        inv = 1.0 / jnp.maximum(cntb[...], 1.0)
        final[...] = (jnp.dot(pooled[...] * inv, wlin[...],
                              preferred_element_type=f32) + blin[...])


def _l2_with_i(*args):
    _layer2_body(pl.program_id(0), *args)


def _l1_with_i(*args):
    _layer1_body(*args)


_row = pl.BlockSpec((RB, D), lambda i: (i, 0))
_col1 = pl.BlockSpec((RB, 1), lambda i: (i, 0))
_wfull = pl.BlockSpec((D, D), lambda i: (0, 0))
_bfull = pl.BlockSpec((1, D), lambda i: (0, 0))

_layer1 = pl.pallas_call(
    _l1_with_i,
    grid=(GB,),
    in_specs=[_row, _row, _row, _col1, _col1, _col1, _row, _row,
              _wfull, _wfull, _wfull, _wfull, _wfull, _bfull, _bfull],
    out_specs=[_row, _row],
    out_shape=[jax.ShapeDtypeStruct((NP, D), jnp.float32),
               jax.ShapeDtypeStruct((NP, D), jnp.float32)],
    compiler_params=pltpu.CompilerParams(
        dimension_semantics=("arbitrary",)),
)

_layer2 = pl.pallas_call(
    _l2_with_i,
    grid=(GB,),
    in_specs=[_row, _row, _col1, _col1, _row,
              pl.BlockSpec((1, 1, RB), lambda i: (i, 0, 0)),
              _wfull, _wfull, _wfull, _bfull,
              pl.BlockSpec((D, C), lambda i: (0, 0)),
              pl.BlockSpec((1, C), lambda i: (0, 0))],
    out_specs=[pl.BlockSpec((G, D), lambda i: (0, 0)),
               pl.BlockSpec((G, D), lambda i: (0, 0)),
               pl.BlockSpec((G, C), lambda i: (0, 0))],
    out_shape=[jax.ShapeDtypeStruct((G, D), jnp.float32),
               jax.ShapeDtypeStruct((G, D), jnp.float32),
               jax.ShapeDtypeStruct((G, C), jnp.float32)],
    compiler_params=pltpu.CompilerParams(
        dimension_semantics=("arbitrary",)),
)


def _pad_edges(ei):
    src = jnp.concatenate([ei[0], jnp.zeros((EP - E,), jnp.int32)])
    dst = jnp.concatenate([ei[1], jnp.full((EP - E,), 1 << 28, jnp.int32)])
    return src.reshape(NS, EPT // 16, 16), dst.reshape(NS, EPT // 16, 16)


def kernel(x_paper, x_author, edge_index_cites, edge_index_writes,
           edge_index_rev, batch, Wl1c, bl1c, Wr1c, Wl1w, bl1w, Wr1w,
           Wl1r, bl1r, Wr1r, Wl2c, bl2c, Wr2c, Wl2w, bl2w, Wr2w,
           Wl2r, bl2r, Wr2r, Wlin, blin):
    padn = jnp.zeros((NP - N, D), jnp.float32)
    xp = jnp.concatenate([x_paper, padn])
    xa = jnp.concatenate([x_author, padn])
    sc_, dc_ = _pad_edges(edge_index_cites)
    sw_, dw_ = _pad_edges(edge_index_writes)
    sr_, dr_ = _pad_edges(edge_index_rev)
    bat = jnp.concatenate([batch, jnp.full((NP - N,), G, jnp.int32)])
    bat = bat.reshape(GB, 1, RB)

    aggc, aggw, aggr, cc, cw, cr = _agg3(xp, xa, xp,
                                         sc_, dc_, sw_, dw_, sr_, dr_)
    cc = cc.reshape(NP, 1)
    cw = cw.reshape(NP, 1)
    cr = cr.reshape(NP, 1)

    xp1, xa1 = _layer1(aggc, aggw, aggr, cc, cw, cr, xp, xa,
                       Wl1c, Wl1w, Wl1r, (Wr1c + Wr1w), Wr1r,
                       (bl1c + bl1w).reshape(1, D), bl1r.reshape(1, D))

    aggc2, aggw2 = _agg2(xp1, xa1, sc_, dc_, sw_, dw_)

    _, _, final = _layer2(aggc2, aggw2, cc, cw, xp1, bat,
                          Wl2c, Wl2w, (Wr2c + Wr2w),
                          (bl2c + bl2w).reshape(1, D),
                          Wlin, blin.reshape(1, C))
    return final


# speedup vs baseline: 3.3262x; 1.0847x over previous
import functools

import jax
import jax.numpy as jnp
from jax import lax
from jax.experimental import pallas as pl
from jax.experimental.pallas import tpu as pltpu
from jax.experimental.pallas import tpu_sc as plsc

N = 50000
E = 200000
D = 128
G = 64
C = 32

NC = 2
NS = 16
NPASS = 4
RNG = 6272
NP = 8 * RNG
ACC = RNG + 256
EPT = 12544
EP = NS * EPT
CAP = 12800
RB = 256
GB = NP // RB


def _agg_body(nrels, with_counts, *refs):
    pos = 0
    tabs = refs[pos:pos + nrels]; pos += nrels
    edges = refs[pos:pos + 2 * nrels]; pos += 2 * nrels
    aggs = refs[pos:pos + nrels]; pos += nrels
    cnts = refs[pos:pos + nrels] if with_counts else ()
    if with_counts:
        pos += nrels
    (acc_sp, cnt_sp, srcv, dstv, cg, cs, rowbuf0, rowbuf1, zrow, onesv,
     zcnt, cntv, gsem0, gsem1, ssem0, ssem1) = refs[pos:]
    rowbufs = (rowbuf0, rowbuf1)
    gsems = (gsem0, gsem1)
    ssems = (ssem0, ssem1)

    core = lax.axis_index("c")
    sub = lax.axis_index("s")
    zero16 = jnp.zeros((16,), jnp.float32)
    one16 = jnp.ones((16,), jnp.float32)
    iota16 = lax.iota(jnp.int32, 16)

    @pl.loop(0, ACC // NS // 16 + 1)
    def _(k):
        zcnt[pl.ds(k * 16, 16)] = zero16

    @pl.loop(0, 8)
    def _(k):
        onesv[pl.ds(k * 16, 16)] = one16

    @pl.loop(0, 512)
    def _(k):
        zrow[k >> 3, pl.ds((k & 7) * 16, 16)] = zero16

    trash_s = RNG + ((sub * 16 + iota16) & 255)
    trash_g = sub * 16 + iota16

    for r in range(nrels):
        tab = tabs[r]
        src_h, dst_h = edges[2 * r], edges[2 * r + 1]
        for p in range(NPASS):
            rid = 2 * p + core
            lo = rid * RNG

            zbase = sub * (ACC // NS)
            for k in range(6):
                pltpu.sync_copy(zrow,
                                acc_sp.at[pl.ds(zbase + 64 * k, 64)])
            pltpu.sync_copy(zrow.at[pl.ds(0, ACC // NS - 384)],
                            acc_sp.at[pl.ds(zbase + 384, ACC // NS - 384)])
            if with_counts:
                pltpu.sync_copy(zcnt.at[pl.ds(0, ACC // NS)],
                                cnt_sp.at[pl.ds(zbase, ACC // NS)])

            plsc.subcore_barrier()

            def _chunk(c, cnt):
                pltpu.sync_copy(src_h.at[sub, pl.ds(c * 112, 112)], srcv)
                pltpu.sync_copy(dst_h.at[sub, pl.ds(c * 112, 112)], dstv)

                def _scan(q, cnt):
                    ss, ds_, ms, cums = [], [], [], []
                    for jj in range(4):
                        j = q * 4 + jj
                        s16 = srcv[j]
                        d16 = dstv[j]
                        m = (d16 >= lo) & (d16 < lo + RNG)
                        ss.append(s16)
                        ds_.append(d16)
                        ms.append(m)
                        cums.append(plsc.cumsum(m.astype(jnp.int32)))
                    for jj in range(4):
                        pos = cnt + cums[jj] - 1
                        plsc.store_scatter(cg, [pos], ss[jj], mask=ms[jj])
                        plsc.store_scatter(cs, [pos], ds_[jj] - lo,
                                           mask=ms[jj])
                        cnt = cnt + cums[jj][15]
                    return cnt

                return lax.fori_loop(0, 28, _scan, cnt)

            cnt = lax.fori_loop(0, 7, _chunk, jnp.int32(0))

            for k in range(8):
                tpos = cnt + 16 * k + iota16
                plsc.store_scatter(cg, [tpos], trash_g)
                plsc.store_scatter(cs, [tpos], trash_s)

            nb = (cnt + 127) >> 7

            def _drain(u):
                dummy = cs.at[pl.ds(0, 128)]
                pltpu.make_async_copy(rowbufs[u], acc_sp.at[dummy],
                                      ssems[u]).wait()
                if with_counts:
                    pltpu.make_async_copy(onesv, cnt_sp.at[dummy],
                                          ssems[u]).wait()

            def _half(h, _):
                for u in range(2):
                    b = h * 2 + u

                    @pl.when(b < nb)
                    def _():
                        @pl.when(b >= 2)
                        def _():
                            _drain(u)
                        gslice = cg.at[pl.ds(b * 128, 128)]
                        pltpu.async_copy(tab.at[gslice], rowbufs[u],
                                         gsems[u])
                for u in range(2):
                    b = h * 2 + u

                    @pl.when(b < nb)
                    def _():
                        gslice = cg.at[pl.ds(b * 128, 128)]
                        sslice = cs.at[pl.ds(b * 128, 128)]
                        pltpu.make_async_copy(tab.at[gslice], rowbufs[u],
                                              gsems[u]).wait()
                        pltpu.async_copy(rowbufs[u], acc_sp.at[sslice],
                                         ssems[u], add=True)
                        if with_counts:
                            pltpu.async_copy(onesv, cnt_sp.at[sslice],
                                             ssems[u], add=True)
                return 0

            lax.fori_loop(0, (nb + 1) >> 1, _half, 0)
            for u in range(2):
                @pl.when(nb > u)
                def _():
                    _drain(u)

            plsc.subcore_barrier()
            wbase = sub * (RNG // NS)
            pltpu.sync_copy(acc_sp.at[pl.ds(wbase, RNG // NS)],
                            aggs[r].at[pl.ds(lo + wbase, RNG // NS)])
            if with_counts:
                pltpu.sync_copy(
                    cnt_sp.at[pl.ds(pl.multiple_of(wbase, 8), RNG // NS)],
                    cntv)
                pltpu.sync_copy(cntv, cnts[r].at[rid * NS + sub])
            plsc.subcore_barrier()


def _make_agg(nrels, with_counts):
    outs = [jax.ShapeDtypeStruct((NP, D), jnp.float32) for _ in range(nrels)]
    if with_counts:
        outs += [jax.ShapeDtypeStruct((2 * NPASS * NS, RNG // NS),
                                      jnp.float32)
                 for _ in range(nrels)]
    mesh = plsc.VectorSubcoreMesh(core_axis_name="c", subcore_axis_name="s",
                                  num_cores=NC, num_subcores=NS)
    return pl.kernel(
        functools.partial(_agg_body, nrels, with_counts),
        out_type=tuple(outs),
        mesh=mesh,
        scratch_types=[
            pltpu.VMEM_SHARED((ACC, D), jnp.float32),
            pltpu.VMEM_SHARED((ACC,), jnp.float32),
            pltpu.VMEM((112, 16), jnp.int32),
            pltpu.VMEM((112, 16), jnp.int32),
            pltpu.VMEM((CAP,), jnp.int32),
            pltpu.VMEM((CAP,), jnp.int32),
            pltpu.VMEM((128, D), jnp.float32),
            pltpu.VMEM((128, D), jnp.float32),
            pltpu.VMEM((64, D), jnp.float32),
            pltpu.VMEM((128,), jnp.float32),
            pltpu.VMEM((800,), jnp.float32),
            pltpu.VMEM((RNG // NS,), jnp.float32),
            pltpu.SemaphoreType.DMA,
            pltpu.SemaphoreType.DMA,
            pltpu.SemaphoreType.DMA,
            pltpu.SemaphoreType.DMA,
        ],
        compiler_params=pltpu.CompilerParams(use_tc_tiling_on_sc=False,
                                             needs_layout_passes=False),
        name=f"sc_agg{nrels}",
    )


_agg3 = _make_agg(3, True)
_agg2 = _make_agg(2, False)


def _layer1_body(aggc, aggw, aggr, cc, cw, cr, xp, xa,
                 wlc, wlw, wlr, wrp, wra, bp, ba, xp1, xa1):
    invc = 1.0 / jnp.maximum(cc[...], 1.0)
    invw = 1.0 / jnp.maximum(cw[...], 1.0)
    invr = 1.0 / jnp.maximum(cr[...], 1.0)
    f32 = jnp.float32
    hp = (jnp.dot(aggc[...] * invc, wlc[...], preferred_element_type=f32)
          + jnp.dot(aggw[...] * invw, wlw[...], preferred_element_type=f32)
          + jnp.dot(xp[...], wrp[...], preferred_element_type=f32) + bp[...])
    ha = (jnp.dot(aggr[...] * invr, wlr[...], preferred_element_type=f32)
          + jnp.dot(xa[...], wra[...], preferred_element_type=f32) + ba[...])
    xp1[...] = jnp.maximum(hp, 0.0)
    xa1[...] = jnp.maximum(ha, 0.0)


def _layer2_body(i, aggc, aggw, cc, cw, xp1, bat, wlc, wlw, wrp, bp,
                 wlin, blin, pooled, cntb, final):
    invc = 1.0 / jnp.maximum(cc[...], 1.0)
    invw = 1.0 / jnp.maximum(cw[...], 1.0)
    f32 = jnp.float32
    hp = (jnp.dot(aggc[...] * invc, wlc[...], preferred_element_type=f32)
          + jnp.dot(aggw[...] * invw, wlw[...], preferred_element_type=f32)
          + jnp.dot(xp1[...], wrp[...], preferred_element_type=f32) + bp[...])
    xp2 = jnp.maximum(hp, 0.0)
    xp2 = jnp.where(xp2 < jnp.float32(1e30), xp2, jnp.float32(0.0))
    bb = bat[0]
    oh = (lax.broadcasted_iota(jnp.int32, (G, RB), 0)
          == jnp.broadcast_to(bb, (G, RB))).astype(f32)

    @pl.when(i == 0)
    def _():
        pooled[...] = jnp.zeros((G, D), f32)
        cntb[...] = jnp.zeros((G, D), f32)

    pooled[...] += jnp.dot(oh, xp2, preferred_element_type=f32)
    cntb[...] += jnp.broadcast_to(jnp.sum(oh, axis=1, keepdims=True), (G, D))

    @pl.when(i == GB - 1)
    def _():
        inv = 1.0 / jnp.maximum(cntb[...], 1.0)
        final[...] = (jnp.dot(pooled[...] * inv, wlin[...],
                              preferred_element_type=f32) + blin[...])


def _l2_with_i(*args):
    _layer2_body(pl.program_id(0), *args)


def _l1_with_i(*args):
    _layer1_body(*args)


_row = pl.BlockSpec((RB, D), lambda i: (i, 0))
_col1 = pl.BlockSpec((RB, 1), lambda i: (i, 0))
_wfull = pl.BlockSpec((D, D), lambda i: (0, 0))
_bfull = pl.BlockSpec((1, D), lambda i: (0, 0))

_layer1 = pl.pallas_call(
    _l1_with_i,
    grid=(GB,),
    in_specs=[_row, _row, _row, _col1, _col1, _col1, _row, _row,
              _wfull, _wfull, _wfull, _wfull, _wfull, _bfull, _bfull],
    out_specs=[_row, _row],
    out_shape=[jax.ShapeDtypeStruct((NP, D), jnp.float32),
               jax.ShapeDtypeStruct((NP, D), jnp.float32)],
    compiler_params=pltpu.CompilerParams(
        dimension_semantics=("arbitrary",)),
)

_layer2 = pl.pallas_call(
    _l2_with_i,
    grid=(GB,),
    in_specs=[_row, _row, _col1, _col1, _row,
              pl.BlockSpec((1, 1, RB), lambda i: (i, 0, 0)),
              _wfull, _wfull, _wfull, _bfull,
              pl.BlockSpec((D, C), lambda i: (0, 0)),
              pl.BlockSpec((1, C), lambda i: (0, 0))],
    out_specs=[pl.BlockSpec((G, D), lambda i: (0, 0)),
               pl.BlockSpec((G, D), lambda i: (0, 0)),
               pl.BlockSpec((G, C), lambda i: (0, 0))],
    out_shape=[jax.ShapeDtypeStruct((G, D), jnp.float32),
               jax.ShapeDtypeStruct((G, D), jnp.float32),
               jax.ShapeDtypeStruct((G, C), jnp.float32)],
    compiler_params=pltpu.CompilerParams(
        dimension_semantics=("arbitrary",)),
)


def _pad_edges(ei):
    src = jnp.concatenate([ei[0], jnp.zeros((EP - E,), jnp.int32)])
    dst = jnp.concatenate([ei[1], jnp.full((EP - E,), 1 << 28, jnp.int32)])
    return src.reshape(NS, EPT // 16, 16), dst.reshape(NS, EPT // 16, 16)


def kernel(x_paper, x_author, edge_index_cites, edge_index_writes,
           edge_index_rev, batch, Wl1c, bl1c, Wr1c, Wl1w, bl1w, Wr1w,
           Wl1r, bl1r, Wr1r, Wl2c, bl2c, Wr2c, Wl2w, bl2w, Wr2w,
           Wl2r, bl2r, Wr2r, Wlin, blin):
    xp = x_paper
    xa = x_author
    sc_, dc_ = _pad_edges(edge_index_cites)
    sw_, dw_ = _pad_edges(edge_index_writes)
    sr_, dr_ = _pad_edges(edge_index_rev)
    bat = jnp.concatenate([batch, jnp.full((NP - N,), G, jnp.int32)])
    bat = bat.reshape(GB, 1, RB)

    aggc, aggw, aggr, cc, cw, cr = _agg3(xp, xa, xp,
                                         sc_, dc_, sw_, dw_, sr_, dr_)
    cc = cc.reshape(NP, 1)
    cw = cw.reshape(NP, 1)
    cr = cr.reshape(NP, 1)

    xp1, xa1 = _layer1(aggc, aggw, aggr, cc, cw, cr, xp, xa,
                       Wl1c, Wl1w, Wl1r, (Wr1c + Wr1w), Wr1r,
                       (bl1c + bl1w).reshape(1, D), bl1r.reshape(1, D))

    aggc2, aggw2 = _agg2(xp1, xa1, sc_, dc_, sw_, dw_)

    _, _, final = _layer2(aggc2, aggw2, cc, cw, xp1, bat,
                          Wl2c, Wl2w, (Wr2c + Wr2w),
                          (bl2c + bl2w).reshape(1, D),
                          Wlin, blin.reshape(1, C))
    return final


# speedup vs baseline: 3.6484x; 1.0969x over previous
import functools

import jax
import jax.numpy as jnp
from jax import lax
from jax.experimental import pallas as pl
from jax.experimental.pallas import tpu as pltpu
from jax.experimental.pallas import tpu_sc as plsc

N = 50000
E = 200000
D = 128
G = 64
C = 32

NC = 2
NS = 16
NPASS = 4
RNG = 6272
NP = 8 * RNG
ACC = RNG + 256
EPT = 12544
EP = NS * EPT
CAP = 12800
RB = 512
GB = NP // RB


def _agg_body(nrels, with_counts, *refs):
    pos = 0
    tabs = refs[pos:pos + nrels]; pos += nrels
    edges = refs[pos:pos + 2 * nrels]; pos += 2 * nrels
    aggs = refs[pos:pos + nrels]; pos += nrels
    cnts = refs[pos:pos + nrels] if with_counts else ()
    if with_counts:
        pos += nrels
    (acc_sp, cnt_sp, srcv, dstv, cg, cs, rowbuf0, rowbuf1, zrow, onesv,
     zcnt, cntv, gsem0, gsem1, ssem0, ssem1) = refs[pos:]
    rowbufs = (rowbuf0, rowbuf1)
    gsems = (gsem0, gsem1)
    ssems = (ssem0, ssem1)

    core = lax.axis_index("c")
    sub = lax.axis_index("s")
    zero16 = jnp.zeros((16,), jnp.float32)
    one16 = jnp.ones((16,), jnp.float32)
    iota16 = lax.iota(jnp.int32, 16)

    @pl.loop(0, ACC // NS // 16 + 1)
    def _(k):
        zcnt[pl.ds(k * 16, 16)] = zero16

    @pl.loop(0, 8)
    def _(k):
        onesv[pl.ds(k * 16, 16)] = one16

    @pl.loop(0, 512)
    def _(k):
        zrow[k >> 3, pl.ds((k & 7) * 16, 16)] = zero16

    trash_s = RNG + ((sub * 16 + iota16) & 255)
    trash_g = sub * 16 + iota16

    for r in range(nrels):
        tab = tabs[r]
        src_h, dst_h = edges[2 * r], edges[2 * r + 1]
        for p in range(NPASS):
            rid = 2 * p + core
            lo = rid * RNG

            zbase = sub * (ACC // NS)
            for k in range(6):
                pltpu.sync_copy(zrow,
                                acc_sp.at[pl.ds(zbase + 64 * k, 64)])
            pltpu.sync_copy(zrow.at[pl.ds(0, ACC // NS - 384)],
                            acc_sp.at[pl.ds(zbase + 384, ACC // NS - 384)])
            if with_counts:
                pltpu.sync_copy(zcnt.at[pl.ds(0, ACC // NS)],
                                cnt_sp.at[pl.ds(zbase, ACC // NS)])

            plsc.subcore_barrier()

            def _chunk(c, cnt):
                pltpu.sync_copy(src_h.at[sub, pl.ds(c * 112, 112)], srcv)
                pltpu.sync_copy(dst_h.at[sub, pl.ds(c * 112, 112)], dstv)

                def _scan(q, cnt):
                    ss, ds_, ms, cums = [], [], [], []
                    for jj in range(4):
                        j = q * 4 + jj
                        s16 = srcv[j]
                        d16 = dstv[j]
                        m = (d16 >= lo) & (d16 < lo + RNG)
                        ss.append(s16)
                        ds_.append(d16)
                        ms.append(m)
                        cums.append(plsc.cumsum(m.astype(jnp.int32)))
                    for jj in range(4):
                        pos = cnt + cums[jj] - 1
                        plsc.store_scatter(cg, [pos], ss[jj], mask=ms[jj])
                        plsc.store_scatter(cs, [pos], ds_[jj] - lo,
                                           mask=ms[jj])
                        cnt = cnt + cums[jj][15]
                    return cnt

                return lax.fori_loop(0, 28, _scan, cnt)

            cnt = lax.fori_loop(0, 7, _chunk, jnp.int32(0))

            for k in range(8):
                tpos = cnt + 16 * k + iota16
                plsc.store_scatter(cg, [tpos], trash_g)
                plsc.store_scatter(cs, [tpos], trash_s)

            nb = (cnt + 127) >> 7

            def _drain(u):
                dummy = cs.at[pl.ds(0, 128)]
                pltpu.make_async_copy(rowbufs[u], acc_sp.at[dummy],
                                      ssems[u]).wait()
                if with_counts:
                    pltpu.make_async_copy(onesv, cnt_sp.at[dummy],
                                          ssems[u]).wait()

            def _half(h, _):
                for u in range(2):
                    b = h * 2 + u

                    @pl.when(b < nb)
                    def _():
                        @pl.when(b >= 2)
                        def _():
                            _drain(u)
                        gslice = cg.at[pl.ds(b * 128, 128)]
                        pltpu.async_copy(tab.at[gslice], rowbufs[u],
                                         gsems[u])
                for u in range(2):
                    b = h * 2 + u

                    @pl.when(b < nb)
                    def _():
                        gslice = cg.at[pl.ds(b * 128, 128)]
                        sslice = cs.at[pl.ds(b * 128, 128)]
                        pltpu.make_async_copy(tab.at[gslice], rowbufs[u],
                                              gsems[u]).wait()
                        pltpu.async_copy(rowbufs[u], acc_sp.at[sslice],
                                         ssems[u], add=True)
                        if with_counts:
                            pltpu.async_copy(onesv, cnt_sp.at[sslice],
                                             ssems[u], add=True)
                return 0

            lax.fori_loop(0, (nb + 1) >> 1, _half, 0)
            for u in range(2):
                @pl.when(nb > u)
                def _():
                    _drain(u)

            plsc.subcore_barrier()
            wbase = sub * (RNG // NS)
            pltpu.sync_copy(acc_sp.at[pl.ds(wbase, RNG // NS)],
                            aggs[r].at[pl.ds(lo + wbase, RNG // NS)])
            if with_counts:
                pltpu.sync_copy(
                    cnt_sp.at[pl.ds(pl.multiple_of(wbase, 8), RNG // NS)],
                    cntv)
                pltpu.sync_copy(cntv, cnts[r].at[rid * NS + sub])
            plsc.subcore_barrier()


def _make_agg(nrels, with_counts):
    outs = [jax.ShapeDtypeStruct((NP, D), jnp.float32) for _ in range(nrels)]
    if with_counts:
        outs += [jax.ShapeDtypeStruct((2 * NPASS * NS, RNG // NS),
                                      jnp.float32)
                 for _ in range(nrels)]
    mesh = plsc.VectorSubcoreMesh(core_axis_name="c", subcore_axis_name="s",
                                  num_cores=NC, num_subcores=NS)
    return pl.kernel(
        functools.partial(_agg_body, nrels, with_counts),
        out_type=tuple(outs),
        mesh=mesh,
        scratch_types=[
            pltpu.VMEM_SHARED((ACC, D), jnp.float32),
            pltpu.VMEM_SHARED((ACC,), jnp.float32),
            pltpu.VMEM((112, 16), jnp.int32),
            pltpu.VMEM((112, 16), jnp.int32),
            pltpu.VMEM((CAP,), jnp.int32),
            pltpu.VMEM((CAP,), jnp.int32),
            pltpu.VMEM((128, D), jnp.float32),
            pltpu.VMEM((128, D), jnp.float32),
            pltpu.VMEM((64, D), jnp.float32),
            pltpu.VMEM((128,), jnp.float32),
            pltpu.VMEM((800,), jnp.float32),
            pltpu.VMEM((RNG // NS,), jnp.float32),
            pltpu.SemaphoreType.DMA,
            pltpu.SemaphoreType.DMA,
            pltpu.SemaphoreType.DMA,
            pltpu.SemaphoreType.DMA,
        ],
        compiler_params=pltpu.CompilerParams(use_tc_tiling_on_sc=False,
                                             needs_layout_passes=False),
        name=f"sc_agg{nrels}",
    )


_agg3 = _make_agg(3, True)
_agg2 = _make_agg(2, False)


def _layer1_body(aggc, aggw, aggr, cc, cw, cr, xp, xa,
                 wlc, wlw, wlr, wrp, wra, bp, ba, xp1, xa1):
    invc = 1.0 / jnp.maximum(cc[...], 1.0)
    invw = 1.0 / jnp.maximum(cw[...], 1.0)
    invr = 1.0 / jnp.maximum(cr[...], 1.0)
    f32 = jnp.float32
    hp = (jnp.dot(aggc[...] * invc, wlc[...], preferred_element_type=f32)
          + jnp.dot(aggw[...] * invw, wlw[...], preferred_element_type=f32)
          + jnp.dot(xp[...], wrp[...], preferred_element_type=f32) + bp[...])
    ha = (jnp.dot(aggr[...] * invr, wlr[...], preferred_element_type=f32)
          + jnp.dot(xa[...], wra[...], preferred_element_type=f32) + ba[...])
    xp1[...] = jnp.maximum(hp, 0.0)
    xa1[...] = jnp.maximum(ha, 0.0)


def _layer2_body(i, aggc, aggw, cc, cw, xp1, bat, wlc, wlw, wrp, bp,
                 wlin, blin, pooled, cntb, final):
    invc = 1.0 / jnp.maximum(cc[...], 1.0)
    invw = 1.0 / jnp.maximum(cw[...], 1.0)
    f32 = jnp.float32
    hp = (jnp.dot(aggc[...] * invc, wlc[...], preferred_element_type=f32)
          + jnp.dot(aggw[...] * invw, wlw[...], preferred_element_type=f32)
          + jnp.dot(xp1[...], wrp[...], preferred_element_type=f32) + bp[...])
    xp2 = jnp.maximum(hp, 0.0)
    xp2 = jnp.where(xp2 < jnp.float32(1e30), xp2, jnp.float32(0.0))
    bb = bat[0]
    oh = (lax.broadcasted_iota(jnp.int32, (G, RB), 0)
          == jnp.broadcast_to(bb, (G, RB))).astype(f32)

    @pl.when(i == 0)
    def _():
        pooled[...] = jnp.zeros((G, D), f32)
        cntb[...] = jnp.zeros((G, D), f32)

    pooled[...] += jnp.dot(oh, xp2, preferred_element_type=f32)
    cntb[...] += jnp.broadcast_to(jnp.sum(oh, axis=1, keepdims=True), (G, D))

    @pl.when(i == GB - 1)
    def _():
        inv = 1.0 / jnp.maximum(cntb[...], 1.0)
        final[...] = (jnp.dot(pooled[...] * inv, wlin[...],
                              preferred_element_type=f32) + blin[...])


def _l2_with_i(*args):
    _layer2_body(pl.program_id(0), *args)


def _l1_with_i(*args):
    _layer1_body(*args)


_row = pl.BlockSpec((RB, D), lambda i: (i, 0))
_col1 = pl.BlockSpec((RB, 1), lambda i: (i, 0))
_wfull = pl.BlockSpec((D, D), lambda i: (0, 0))
_bfull = pl.BlockSpec((1, D), lambda i: (0, 0))

_layer1 = pl.pallas_call(
    _l1_with_i,
    grid=(GB,),
    in_specs=[_row, _row, _row, _col1, _col1, _col1, _row, _row,
              _wfull, _wfull, _wfull, _wfull, _wfull, _bfull, _bfull],
    out_specs=[_row, _row],
    out_shape=[jax.ShapeDtypeStruct((NP, D), jnp.float32),
               jax.ShapeDtypeStruct((NP, D), jnp.float32)],
    compiler_params=pltpu.CompilerParams(
        dimension_semantics=("arbitrary",)),
)

_layer2 = pl.pallas_call(
    _l2_with_i,
    grid=(GB,),
    in_specs=[_row, _row, _col1, _col1, _row,
              pl.BlockSpec((1, 1, RB), lambda i: (i, 0, 0)),
              _wfull, _wfull, _wfull, _bfull,
              pl.BlockSpec((D, C), lambda i: (0, 0)),
              pl.BlockSpec((1, C), lambda i: (0, 0))],
    out_specs=[pl.BlockSpec((G, D), lambda i: (0, 0)),
               pl.BlockSpec((G, D), lambda i: (0, 0)),
               pl.BlockSpec((G, C), lambda i: (0, 0))],
    out_shape=[jax.ShapeDtypeStruct((G, D), jnp.float32),
               jax.ShapeDtypeStruct((G, D), jnp.float32),
               jax.ShapeDtypeStruct((G, C), jnp.float32)],
    compiler_params=pltpu.CompilerParams(
        dimension_semantics=("arbitrary",)),
)


def _pad_edges(ei):
    src = jnp.concatenate([ei[0], jnp.zeros((EP - E,), jnp.int32)])
    dst = jnp.concatenate([ei[1], jnp.full((EP - E,), 1 << 28, jnp.int32)])
    return src.reshape(NS, EPT // 16, 16), dst.reshape(NS, EPT // 16, 16)


def kernel(x_paper, x_author, edge_index_cites, edge_index_writes,
           edge_index_rev, batch, Wl1c, bl1c, Wr1c, Wl1w, bl1w, Wr1w,
           Wl1r, bl1r, Wr1r, Wl2c, bl2c, Wr2c, Wl2w, bl2w, Wr2w,
           Wl2r, bl2r, Wr2r, Wlin, blin):
    xp = x_paper
    xa = x_author
    sc_, dc_ = _pad_edges(edge_index_cites)
    sw_, dw_ = _pad_edges(edge_index_writes)
    sr_, dr_ = _pad_edges(edge_index_rev)
    bat = jnp.concatenate([batch, jnp.full((NP - N,), G, jnp.int32)])
    bat = bat.reshape(GB, 1, RB)

    aggc, aggw, aggr, cc, cw, cr = _agg3(xp, xa, xp,
                                         sc_, dc_, sw_, dw_, sr_, dr_)
    cc = cc.reshape(NP, 1)
    cw = cw.reshape(NP, 1)
    cr = cr.reshape(NP, 1)

    xp1, xa1 = _layer1(aggc, aggw, aggr, cc, cw, cr, xp, xa,
                       Wl1c, Wl1w, Wl1r, (Wr1c + Wr1w), Wr1r,
                       (bl1c + bl1w).reshape(1, D), bl1r.reshape(1, D))

    aggc2, aggw2 = _agg2(xp1, xa1, sc_, dc_, sw_, dw_)

    _, _, final = _layer2(aggc2, aggw2, cc, cw, xp1, bat,
                          Wl2c, Wl2w, (Wr2c + Wr2w),
                          (bl2c + bl2w).reshape(1, D),
                          Wlin, blin.reshape(1, C))
    return final


# speedup vs baseline: 3.8356x; 1.0513x over previous
import functools

import jax
import jax.numpy as jnp
from jax import lax
from jax.experimental import pallas as pl
from jax.experimental.pallas import tpu as pltpu
from jax.experimental.pallas import tpu_sc as plsc

N = 50000
E = 200000
D = 128
G = 64
C = 32

NC = 2
NS = 16
NPASS = 4
RNG = 6272
NP = 8 * RNG
ACC = RNG + 256
EPT = 12544
EP = NS * EPT
CAP = 12800
RB = 1024
GB = NP // RB


def _agg_body(nrels, with_counts, *refs):
    pos = 0
    tabs = refs[pos:pos + nrels]; pos += nrels
    edges = refs[pos:pos + 2 * nrels]; pos += 2 * nrels
    aggs = refs[pos:pos + nrels]; pos += nrels
    cnts = refs[pos:pos + nrels] if with_counts else ()
    if with_counts:
        pos += nrels
    (acc_sp, cnt_sp, srcv, dstv, cg, cs, rowbuf0, rowbuf1, zrow, onesv,
     zcnt, cntv, gsem0, gsem1, ssem0, ssem1) = refs[pos:]
    rowbufs = (rowbuf0, rowbuf1)
    gsems = (gsem0, gsem1)
    ssems = (ssem0, ssem1)

    core = lax.axis_index("c")
    sub = lax.axis_index("s")
    zero16 = jnp.zeros((16,), jnp.float32)
    one16 = jnp.ones((16,), jnp.float32)
    iota16 = lax.iota(jnp.int32, 16)

    @pl.loop(0, ACC // NS // 16 + 1)
    def _(k):
        zcnt[pl.ds(k * 16, 16)] = zero16

    @pl.loop(0, 8)
    def _(k):
        onesv[pl.ds(k * 16, 16)] = one16

    @pl.loop(0, 512)
    def _(k):
        zrow[k >> 3, pl.ds((k & 7) * 16, 16)] = zero16

    trash_s = RNG + ((sub * 16 + iota16) & 255)
    trash_g = sub * 16 + iota16

    for r in range(nrels):
        tab = tabs[r]
        src_h, dst_h = edges[2 * r], edges[2 * r + 1]
        for p in range(NPASS):
            rid = 2 * p + core
            lo = rid * RNG

            zbase = sub * (ACC // NS)
            for k in range(6):
                pltpu.sync_copy(zrow,
                                acc_sp.at[pl.ds(zbase + 64 * k, 64)])
            pltpu.sync_copy(zrow.at[pl.ds(0, ACC // NS - 384)],
                            acc_sp.at[pl.ds(zbase + 384, ACC // NS - 384)])
            if with_counts:
                pltpu.sync_copy(zcnt.at[pl.ds(0, ACC // NS)],
                                cnt_sp.at[pl.ds(zbase, ACC // NS)])

            plsc.subcore_barrier()

            def _chunk(c, cnt):
                pltpu.sync_copy(src_h.at[sub, pl.ds(c * 112, 112)], srcv)
                pltpu.sync_copy(dst_h.at[sub, pl.ds(c * 112, 112)], dstv)

                def _scan(q, cnt):
                    ss, ds_, ms, cums = [], [], [], []
                    for jj in range(4):
                        j = q * 4 + jj
                        s16 = srcv[j]
                        d16 = dstv[j]
                        m = (d16 >= lo) & (d16 < lo + RNG)
                        ss.append(s16)
                        ds_.append(d16)
                        ms.append(m)
                        cums.append(plsc.cumsum(m.astype(jnp.int32)))
                    for jj in range(4):
                        pos = cnt + cums[jj] - 1
                        plsc.store_scatter(cg, [pos], ss[jj], mask=ms[jj])
                        plsc.store_scatter(cs, [pos], ds_[jj] - lo,
                                           mask=ms[jj])
                        cnt = cnt + cums[jj][15]
                    return cnt

                return lax.fori_loop(0, 28, _scan, cnt)

            cnt = lax.fori_loop(0, 7, _chunk, jnp.int32(0))

            for k in range(8):
                tpos = cnt + 16 * k + iota16
                plsc.store_scatter(cg, [tpos], trash_g)
                plsc.store_scatter(cs, [tpos], trash_s)

            nb = (cnt + 127) >> 7

            def _drain(u):
                dummy = cs.at[pl.ds(0, 128)]
                pltpu.make_async_copy(rowbufs[u], acc_sp.at[dummy],
                                      ssems[u]).wait()
                if with_counts:
                    pltpu.make_async_copy(onesv, cnt_sp.at[dummy],
                                          ssems[u]).wait()

            def _half(h, _):
                for u in range(2):
                    b = h * 2 + u

                    @pl.when(b < nb)
                    def _():
                        @pl.when(b >= 2)
                        def _():
                            _drain(u)
                        gslice = cg.at[pl.ds(b * 128, 128)]
                        pltpu.async_copy(tab.at[gslice], rowbufs[u],
                                         gsems[u])
                for u in range(2):
                    b = h * 2 + u

                    @pl.when(b < nb)
                    def _():
                        gslice = cg.at[pl.ds(b * 128, 128)]
                        sslice = cs.at[pl.ds(b * 128, 128)]
                        pltpu.make_async_copy(tab.at[gslice], rowbufs[u],
                                              gsems[u]).wait()
                        pltpu.async_copy(rowbufs[u], acc_sp.at[sslice],
                                         ssems[u], add=True)
                        if with_counts:
                            pltpu.async_copy(onesv, cnt_sp.at[sslice],
                                             ssems[u], add=True)
                return 0

            lax.fori_loop(0, (nb + 1) >> 1, _half, 0)
            for u in range(2):
                @pl.when(nb > u)
                def _():
                    _drain(u)

            plsc.subcore_barrier()
            wbase = sub * (RNG // NS)
            pltpu.sync_copy(acc_sp.at[pl.ds(wbase, RNG // NS)],
                            aggs[r].at[pl.ds(lo + wbase, RNG // NS)])
            if with_counts:
                pltpu.sync_copy(
                    cnt_sp.at[pl.ds(pl.multiple_of(wbase, 8), RNG // NS)],
                    cntv)
                pltpu.sync_copy(cntv, cnts[r].at[rid * NS + sub])
            plsc.subcore_barrier()


def _make_agg(nrels, with_counts):
    outs = [jax.ShapeDtypeStruct((NP, D), jnp.float32) for _ in range(nrels)]
    if with_counts:
        outs += [jax.ShapeDtypeStruct((2 * NPASS * NS, RNG // NS),
                                      jnp.float32)
                 for _ in range(nrels)]
    mesh = plsc.VectorSubcoreMesh(core_axis_name="c", subcore_axis_name="s",
                                  num_cores=NC, num_subcores=NS)
    return pl.kernel(
        functools.partial(_agg_body, nrels, with_counts),
        out_type=tuple(outs),
        mesh=mesh,
        scratch_types=[
            pltpu.VMEM_SHARED((ACC, D), jnp.float32),
            pltpu.VMEM_SHARED((ACC,), jnp.float32),
            pltpu.VMEM((112, 16), jnp.int32),
            pltpu.VMEM((112, 16), jnp.int32),
            pltpu.VMEM((CAP,), jnp.int32),
            pltpu.VMEM((CAP,), jnp.int32),
            pltpu.VMEM((128, D), jnp.float32),
            pltpu.VMEM((128, D), jnp.float32),
            pltpu.VMEM((64, D), jnp.float32),
            pltpu.VMEM((128,), jnp.float32),
            pltpu.VMEM((800,), jnp.float32),
            pltpu.VMEM((RNG // NS,), jnp.float32),
            pltpu.SemaphoreType.DMA,
            pltpu.SemaphoreType.DMA,
            pltpu.SemaphoreType.DMA,
            pltpu.SemaphoreType.DMA,
        ],
        compiler_params=pltpu.CompilerParams(use_tc_tiling_on_sc=False,
                                             needs_layout_passes=False),
        name=f"sc_agg{nrels}",
    )


_agg3 = _make_agg(3, True)
_agg2 = _make_agg(2, False)


def _layer1_body(aggc, aggw, aggr, cc, cw, cr, xp, xa,
                 wlc, wlw, wlr, wrp, wra, bp, ba, xp1, xa1):
    invc = 1.0 / jnp.maximum(cc[...], 1.0)
    invw = 1.0 / jnp.maximum(cw[...], 1.0)
    invr = 1.0 / jnp.maximum(cr[...], 1.0)
    f32 = jnp.float32
    hp = (jnp.dot(aggc[...] * invc, wlc[...], preferred_element_type=f32)
          + jnp.dot(aggw[...] * invw, wlw[...], preferred_element_type=f32)
          + jnp.dot(xp[...], wrp[...], preferred_element_type=f32) + bp[...])
    ha = (jnp.dot(aggr[...] * invr, wlr[...], preferred_element_type=f32)
          + jnp.dot(xa[...], wra[...], preferred_element_type=f32) + ba[...])
    xp1[...] = jnp.maximum(hp, 0.0)
    xa1[...] = jnp.maximum(ha, 0.0)


def _layer2_body(i, aggc, aggw, cc, cw, xp1, bat, wlc, wlw, wrp, bp,
                 wlin, blin, pooled, cntb, final):
    invc = 1.0 / jnp.maximum(cc[...], 1.0)
    invw = 1.0 / jnp.maximum(cw[...], 1.0)
    f32 = jnp.float32
    hp = (jnp.dot(aggc[...] * invc, wlc[...], preferred_element_type=f32)
          + jnp.dot(aggw[...] * invw, wlw[...], preferred_element_type=f32)
          + jnp.dot(xp1[...], wrp[...], preferred_element_type=f32) + bp[...])
    xp2 = jnp.maximum(hp, 0.0)
    xp2 = jnp.where(xp2 < jnp.float32(1e30), xp2, jnp.float32(0.0))
    bb = bat[0]
    oh = (lax.broadcasted_iota(jnp.int32, (G, RB), 0)
          == jnp.broadcast_to(bb, (G, RB))).astype(f32)

    @pl.when(i == 0)
    def _():
        pooled[...] = jnp.zeros((G, D), f32)
        cntb[...] = jnp.zeros((G, D), f32)

    pooled[...] += jnp.dot(oh, xp2, preferred_element_type=f32)
    cntb[...] += jnp.broadcast_to(jnp.sum(oh, axis=1, keepdims=True), (G, D))

    @pl.when(i == GB - 1)
    def _():
        inv = 1.0 / jnp.maximum(cntb[...], 1.0)
        final[...] = (jnp.dot(pooled[...] * inv, wlin[...],
                              preferred_element_type=f32) + blin[...])


def _l2_with_i(*args):
    _layer2_body(pl.program_id(0), *args)


def _l1_with_i(*args):
    _layer1_body(*args)


_row = pl.BlockSpec((RB, D), lambda i: (i, 0))
_col1 = pl.BlockSpec((RB, 1), lambda i: (i, 0))
_wfull = pl.BlockSpec((D, D), lambda i: (0, 0))
_bfull = pl.BlockSpec((1, D), lambda i: (0, 0))

_layer1 = pl.pallas_call(
    _l1_with_i,
    grid=(GB,),
    in_specs=[_row, _row, _row, _col1, _col1, _col1, _row, _row,
              _wfull, _wfull, _wfull, _wfull, _wfull, _bfull, _bfull],
    out_specs=[_row, _row],
    out_shape=[jax.ShapeDtypeStruct((NP, D), jnp.float32),
               jax.ShapeDtypeStruct((NP, D), jnp.float32)],
    compiler_params=pltpu.CompilerParams(
        dimension_semantics=("arbitrary",)),
)

_layer2 = pl.pallas_call(
    _l2_with_i,
    grid=(GB,),
    in_specs=[_row, _row, _col1, _col1, _row,
              pl.BlockSpec((1, 1, RB), lambda i: (i, 0, 0)),
              _wfull, _wfull, _wfull, _bfull,
              pl.BlockSpec((D, C), lambda i: (0, 0)),
              pl.BlockSpec((1, C), lambda i: (0, 0))],
    out_specs=[pl.BlockSpec((G, D), lambda i: (0, 0)),
               pl.BlockSpec((G, D), lambda i: (0, 0)),
               pl.BlockSpec((G, C), lambda i: (0, 0))],
    out_shape=[jax.ShapeDtypeStruct((G, D), jnp.float32),
               jax.ShapeDtypeStruct((G, D), jnp.float32),
               jax.ShapeDtypeStruct((G, C), jnp.float32)],
    compiler_params=pltpu.CompilerParams(
        dimension_semantics=("arbitrary",)),
)


def _pad_edges(ei):
    src = jnp.concatenate([ei[0], jnp.zeros((EP - E,), jnp.int32)])
    dst = jnp.concatenate([ei[1], jnp.full((EP - E,), 1 << 28, jnp.int32)])
    return src.reshape(NS, EPT // 16, 16), dst.reshape(NS, EPT // 16, 16)


def kernel(x_paper, x_author, edge_index_cites, edge_index_writes,
           edge_index_rev, batch, Wl1c, bl1c, Wr1c, Wl1w, bl1w, Wr1w,
           Wl1r, bl1r, Wr1r, Wl2c, bl2c, Wr2c, Wl2w, bl2w, Wr2w,
           Wl2r, bl2r, Wr2r, Wlin, blin):
    xp = x_paper
    xa = x_author
    sc_, dc_ = _pad_edges(edge_index_cites)
    sw_, dw_ = _pad_edges(edge_index_writes)
    sr_, dr_ = _pad_edges(edge_index_rev)
    bat = jnp.concatenate([batch, jnp.full((NP - N,), G, jnp.int32)])
    bat = bat.reshape(GB, 1, RB)

    aggc, aggw, aggr, cc, cw, cr = _agg3(xp, xa, xp,
                                         sc_, dc_, sw_, dw_, sr_, dr_)
    cc = cc.reshape(NP, 1)
    cw = cw.reshape(NP, 1)
    cr = cr.reshape(NP, 1)

    xp1, xa1 = _layer1(aggc, aggw, aggr, cc, cw, cr, xp, xa,
                       Wl1c, Wl1w, Wl1r, (Wr1c + Wr1w), Wr1r,
                       (bl1c + bl1w).reshape(1, D), bl1r.reshape(1, D))

    aggc2, aggw2 = _agg2(xp1, xa1, sc_, dc_, sw_, dw_)

    _, _, final = _layer2(aggc2, aggw2, cc, cw, xp1, bat,
                          Wl2c, Wl2w, (Wr2c + Wr2w),
                          (bl2c + bl2w).reshape(1, D),
                          Wlin, blin.reshape(1, C))
    return final


# speedup vs baseline: 3.9090x; 1.0191x over previous
import functools

import jax
import jax.numpy as jnp
from jax import lax
from jax.experimental import pallas as pl
from jax.experimental.pallas import tpu as pltpu
from jax.experimental.pallas import tpu_sc as plsc

N = 50000
E = 200000
D = 128
G = 64
C = 32

NC = 2
NS = 16
NPASS = 4
RNG = 6272
NP = 8 * RNG
ACC = RNG + 256
EPT = 12544
EP = NS * EPT
CAP = 12800
RB = 1792
GB = NP // RB


def _agg_body(nrels, with_counts, *refs):
    pos = 0
    tabs = refs[pos:pos + nrels]; pos += nrels
    edges = refs[pos:pos + 2 * nrels]; pos += 2 * nrels
    aggs = refs[pos:pos + nrels]; pos += nrels
    cnts = refs[pos:pos + nrels] if with_counts else ()
    if with_counts:
        pos += nrels
    (acc_sp, cnt_sp, srcv, dstv, cg, cs, rowbuf0, rowbuf1, zrow, onesv,
     zcnt, cntv, gsem0, gsem1, ssem0, ssem1) = refs[pos:]
    rowbufs = (rowbuf0, rowbuf1)
    gsems = (gsem0, gsem1)
    ssems = (ssem0, ssem1)

    core = lax.axis_index("c")
    sub = lax.axis_index("s")
    zero16 = jnp.zeros((16,), jnp.float32)
    one16 = jnp.ones((16,), jnp.float32)
    iota16 = lax.iota(jnp.int32, 16)

    @pl.loop(0, ACC // NS // 16 + 1)
    def _(k):
        zcnt[pl.ds(k * 16, 16)] = zero16

    @pl.loop(0, 8)
    def _(k):
        onesv[pl.ds(k * 16, 16)] = one16

    @pl.loop(0, 512)
    def _(k):
        zrow[k >> 3, pl.ds((k & 7) * 16, 16)] = zero16

    trash_s = RNG + ((sub * 16 + iota16) & 255)
    trash_g = sub * 16 + iota16

    for r in range(nrels):
        tab = tabs[r]
        src_h, dst_h = edges[2 * r], edges[2 * r + 1]
        for p in range(NPASS):
            rid = 2 * p + core
            lo = rid * RNG

            zbase = sub * (ACC // NS)
            for k in range(6):
                pltpu.sync_copy(zrow,
                                acc_sp.at[pl.ds(zbase + 64 * k, 64)])
            pltpu.sync_copy(zrow.at[pl.ds(0, ACC // NS - 384)],
                            acc_sp.at[pl.ds(zbase + 384, ACC // NS - 384)])
            if with_counts:
                pltpu.sync_copy(zcnt.at[pl.ds(0, ACC // NS)],
                                cnt_sp.at[pl.ds(zbase, ACC // NS)])

            plsc.subcore_barrier()

            def _chunk(c, cnt):
                pltpu.sync_copy(src_h.at[sub, pl.ds(c * 112, 112)], srcv)
                pltpu.sync_copy(dst_h.at[sub, pl.ds(c * 112, 112)], dstv)

                def _scan(q, cnt):
                    ss, ds_, ms, cums = [], [], [], []
                    for jj in range(4):
                        j = q * 4 + jj
                        s16 = srcv[j]
                        d16 = dstv[j]
                        m = (d16 >= lo) & (d16 < lo + RNG)
                        ss.append(s16)
                        ds_.append(d16)
                        ms.append(m)
                        cums.append(plsc.cumsum(m.astype(jnp.int32)))
                    for jj in range(4):
                        pos = cnt + cums[jj] - 1
                        plsc.store_scatter(cg, [pos], ss[jj], mask=ms[jj])
                        plsc.store_scatter(cs, [pos], ds_[jj] - lo,
                                           mask=ms[jj])
                        cnt = cnt + cums[jj][15]
                    return cnt

                return lax.fori_loop(0, 28, _scan, cnt)

            cnt = lax.fori_loop(0, 7, _chunk, jnp.int32(0))

            for k in range(8):
                tpos = cnt + 16 * k + iota16
                plsc.store_scatter(cg, [tpos], trash_g)
                plsc.store_scatter(cs, [tpos], trash_s)

            nb = (cnt + 127) >> 7

            def _drain(u):
                dummy = cs.at[pl.ds(0, 128)]
                pltpu.make_async_copy(rowbufs[u], acc_sp.at[dummy],
                                      ssems[u]).wait()
                if with_counts:
                    pltpu.make_async_copy(onesv, cnt_sp.at[dummy],
                                          ssems[u]).wait()

            def _half(h, _):
                for u in range(2):
                    b = h * 2 + u

                    @pl.when(b < nb)
                    def _():
                        @pl.when(b >= 2)
                        def _():
                            _drain(u)
                        gslice = cg.at[pl.ds(b * 128, 128)]
                        pltpu.async_copy(tab.at[gslice], rowbufs[u],
                                         gsems[u])
                for u in range(2):
                    b = h * 2 + u

                    @pl.when(b < nb)
                    def _():
                        gslice = cg.at[pl.ds(b * 128, 128)]
                        sslice = cs.at[pl.ds(b * 128, 128)]
                        pltpu.make_async_copy(tab.at[gslice], rowbufs[u],
                                              gsems[u]).wait()
                        pltpu.async_copy(rowbufs[u], acc_sp.at[sslice],
                                         ssems[u], add=True)
                        if with_counts:
                            pltpu.async_copy(onesv, cnt_sp.at[sslice],
                                             ssems[u], add=True)
                return 0

            lax.fori_loop(0, (nb + 1) >> 1, _half, 0)
            for u in range(2):
                @pl.when(nb > u)
                def _():
                    _drain(u)

            plsc.subcore_barrier()
            wbase = sub * (RNG // NS)
            pltpu.sync_copy(acc_sp.at[pl.ds(wbase, RNG // NS)],
                            aggs[r].at[pl.ds(lo + wbase, RNG // NS)])
            if with_counts:
                pltpu.sync_copy(
                    cnt_sp.at[pl.ds(pl.multiple_of(wbase, 8), RNG // NS)],
                    cntv)
                pltpu.sync_copy(cntv, cnts[r].at[rid * NS + sub])
            plsc.subcore_barrier()


def _make_agg(nrels, with_counts):
    outs = [jax.ShapeDtypeStruct((NP, D), jnp.float32) for _ in range(nrels)]
    if with_counts:
        outs += [jax.ShapeDtypeStruct((2 * NPASS * NS, RNG // NS),
                                      jnp.float32)
                 for _ in range(nrels)]
    mesh = plsc.VectorSubcoreMesh(core_axis_name="c", subcore_axis_name="s",
                                  num_cores=NC, num_subcores=NS)
    return pl.kernel(
        functools.partial(_agg_body, nrels, with_counts),
        out_type=tuple(outs),
        mesh=mesh,
        scratch_types=[
            pltpu.VMEM_SHARED((ACC, D), jnp.float32),
            pltpu.VMEM_SHARED((ACC,), jnp.float32),
            pltpu.VMEM((112, 16), jnp.int32),
            pltpu.VMEM((112, 16), jnp.int32),
            pltpu.VMEM((CAP,), jnp.int32),
            pltpu.VMEM((CAP,), jnp.int32),
            pltpu.VMEM((128, D), jnp.float32),
            pltpu.VMEM((128, D), jnp.float32),
            pltpu.VMEM((64, D), jnp.float32),
            pltpu.VMEM((128,), jnp.float32),
            pltpu.VMEM((800,), jnp.float32),
            pltpu.VMEM((RNG // NS,), jnp.float32),
            pltpu.SemaphoreType.DMA,
            pltpu.SemaphoreType.DMA,
            pltpu.SemaphoreType.DMA,
            pltpu.SemaphoreType.DMA,
        ],
        compiler_params=pltpu.CompilerParams(use_tc_tiling_on_sc=False,
                                             needs_layout_passes=False),
        name=f"sc_agg{nrels}",
    )


_agg3 = _make_agg(3, True)
_agg2 = _make_agg(2, False)


def _layer1_body(aggc, aggw, aggr, cc, cw, cr, xp, xa,
                 wlc, wlw, wlr, wrp, wra, bp, ba, xp1, xa1):
    invc = 1.0 / jnp.maximum(cc[...], 1.0)
    invw = 1.0 / jnp.maximum(cw[...], 1.0)
    invr = 1.0 / jnp.maximum(cr[...], 1.0)
    f32 = jnp.float32
    hp = (jnp.dot(aggc[...] * invc, wlc[...], preferred_element_type=f32)
          + jnp.dot(aggw[...] * invw, wlw[...], preferred_element_type=f32)
          + jnp.dot(xp[...], wrp[...], preferred_element_type=f32) + bp[...])
    ha = (jnp.dot(aggr[...] * invr, wlr[...], preferred_element_type=f32)
          + jnp.dot(xa[...], wra[...], preferred_element_type=f32) + ba[...])
    xp1[...] = jnp.maximum(hp, 0.0)
    xa1[...] = jnp.maximum(ha, 0.0)


def _layer2_body(i, aggc, aggw, cc, cw, xp1, bat, wlc, wlw, wrp, bp,
                 wlin, blin, pooled, cntb, final):
    invc = 1.0 / jnp.maximum(cc[...], 1.0)
    invw = 1.0 / jnp.maximum(cw[...], 1.0)
    f32 = jnp.float32
    hp = (jnp.dot(aggc[...] * invc, wlc[...], preferred_element_type=f32)
          + jnp.dot(aggw[...] * invw, wlw[...], preferred_element_type=f32)
          + jnp.dot(xp1[...], wrp[...], preferred_element_type=f32) + bp[...])
    xp2 = jnp.maximum(hp, 0.0)
    xp2 = jnp.where(xp2 < jnp.float32(1e30), xp2, jnp.float32(0.0))
    bb = bat[0]
    oh = (lax.broadcasted_iota(jnp.int32, (G, RB), 0)
          == jnp.broadcast_to(bb, (G, RB))).astype(f32)

    @pl.when(i == 0)
    def _():
        pooled[...] = jnp.zeros((G, D), f32)
        cntb[...] = jnp.zeros((G, D), f32)

    pooled[...] += jnp.dot(oh, xp2, preferred_element_type=f32)
    cntb[...] += jnp.broadcast_to(jnp.sum(oh, axis=1, keepdims=True), (G, D))

    @pl.when(i == GB - 1)
    def _():
        inv = 1.0 / jnp.maximum(cntb[...], 1.0)
        final[...] = (jnp.dot(pooled[...] * inv, wlin[...],
                              preferred_element_type=f32) + blin[...])


def _l2_with_i(*args):
    _layer2_body(pl.program_id(0), *args)


def _l1_with_i(*args):
    _layer1_body(*args)


_row = pl.BlockSpec((RB, D), lambda i: (i, 0))
_col1 = pl.BlockSpec((RB, 1), lambda i: (i, 0))
_wfull = pl.BlockSpec((D, D), lambda i: (0, 0))
_bfull = pl.BlockSpec((1, D), lambda i: (0, 0))

_layer1 = pl.pallas_call(
    _l1_with_i,
    grid=(GB,),
    in_specs=[_row, _row, _row, _col1, _col1, _col1, _row, _row,
              _wfull, _wfull, _wfull, _wfull, _wfull, _bfull, _bfull],
    out_specs=[_row, _row],
    out_shape=[jax.ShapeDtypeStruct((NP, D), jnp.float32),
               jax.ShapeDtypeStruct((NP, D), jnp.float32)],
    compiler_params=pltpu.CompilerParams(
        dimension_semantics=("arbitrary",)),
)

_layer2 = pl.pallas_call(
    _l2_with_i,
    grid=(GB,),
    in_specs=[_row, _row, _col1, _col1, _row,
              pl.BlockSpec((1, 1, RB), lambda i: (i, 0, 0)),
              _wfull, _wfull, _wfull, _bfull,
              pl.BlockSpec((D, C), lambda i: (0, 0)),
              pl.BlockSpec((1, C), lambda i: (0, 0))],
    out_specs=[pl.BlockSpec((G, D), lambda i: (0, 0)),
               pl.BlockSpec((G, D), lambda i: (0, 0)),
               pl.BlockSpec((G, C), lambda i: (0, 0))],
    out_shape=[jax.ShapeDtypeStruct((G, D), jnp.float32),
               jax.ShapeDtypeStruct((G, D), jnp.float32),
               jax.ShapeDtypeStruct((G, C), jnp.float32)],
    compiler_params=pltpu.CompilerParams(
        dimension_semantics=("arbitrary",)),
)


def _pad_edges(ei):
    src = jnp.concatenate([ei[0], jnp.zeros((EP - E,), jnp.int32)])
    dst = jnp.concatenate([ei[1], jnp.full((EP - E,), 1 << 28, jnp.int32)])
    return src.reshape(NS, EPT // 16, 16), dst.reshape(NS, EPT // 16, 16)


def kernel(x_paper, x_author, edge_index_cites, edge_index_writes,
           edge_index_rev, batch, Wl1c, bl1c, Wr1c, Wl1w, bl1w, Wr1w,
           Wl1r, bl1r, Wr1r, Wl2c, bl2c, Wr2c, Wl2w, bl2w, Wr2w,
           Wl2r, bl2r, Wr2r, Wlin, blin):
    xp = x_paper
    xa = x_author
    sc_, dc_ = _pad_edges(edge_index_cites)
    sw_, dw_ = _pad_edges(edge_index_writes)
    sr_, dr_ = _pad_edges(edge_index_rev)
    bat = jnp.concatenate([batch, jnp.full((NP - N,), G, jnp.int32)])
    bat = bat.reshape(GB, 1, RB)

    aggc, aggw, aggr, cc, cw, cr = _agg3(xp, xa, xp,
                                         sc_, dc_, sw_, dw_, sr_, dr_)
    cc = cc.reshape(NP, 1)
    cw = cw.reshape(NP, 1)
    cr = cr.reshape(NP, 1)

    xp1, xa1 = _layer1(aggc, aggw, aggr, cc, cw, cr, xp, xa,
                       Wl1c, Wl1w, Wl1r, (Wr1c + Wr1w), Wr1r,
                       (bl1c + bl1w).reshape(1, D), bl1r.reshape(1, D))

    aggc2, aggw2 = _agg2(xp1, xa1, sc_, dc_, sw_, dw_)

    _, _, final = _layer2(aggc2, aggw2, cc, cw, xp1, bat,
                          Wl2c, Wl2w, (Wr2c + Wr2w),
                          (bl2c + bl2w).reshape(1, D),
                          Wlin, blin.reshape(1, C))
    return final


# speedup vs baseline: 4.3778x; 1.1199x over previous
import functools

import jax
import jax.numpy as jnp
from jax import lax
from jax.experimental import pallas as pl
from jax.experimental.pallas import tpu as pltpu
from jax.experimental.pallas import tpu_sc as plsc

N = 50000
E = 200000
D = 128
G = 64
C = 32

NC = 2
NS = 16
NPASS = 4
RNG = 6272
NP = 8 * RNG
ACC = RNG + 256
EPT = 12544
EP = NS * EPT
CAP = 12800
RB = 1792
GB = NP // RB


def _agg_body(nrels, with_counts, *refs):
    pos = 0
    tabs = refs[pos:pos + nrels]; pos += nrels
    edges = refs[pos:pos + 2 * nrels]; pos += 2 * nrels
    aggs = refs[pos:pos + nrels]; pos += nrels
    cnts = refs[pos:pos + nrels] if with_counts else ()
    if with_counts:
        pos += nrels
    (acc_sp, cnt_sp, srcv, dstv, cg, cs, rowbuf0, rowbuf1, zrow, onesv,
     zcnt, cntv, gsem0, gsem1, ssem0, ssem1) = refs[pos:]
    rowbufs = (rowbuf0, rowbuf1)
    gsems = (gsem0, gsem1)
    ssems = (ssem0, ssem1)

    core = lax.axis_index("c")
    sub = lax.axis_index("s")
    zero16 = jnp.zeros((16,), jnp.float32)
    one16 = jnp.ones((16,), jnp.float32)
    iota16 = lax.iota(jnp.int32, 16)

    @pl.loop(0, ACC // NS // 16 + 1)
    def _(k):
        zcnt[pl.ds(k * 16, 16)] = zero16

    @pl.loop(0, 8)
    def _(k):
        onesv[pl.ds(k * 16, 16)] = one16

    @pl.loop(0, 512)
    def _(k):
        zrow[k >> 3, pl.ds((k & 7) * 16, 16)] = zero16

    trash_s = RNG + ((sub * 16 + iota16) & 255)
    trash_g = sub * 16 + iota16

    for r in range(nrels):
        tab = tabs[r]
        src_h, dst_h = edges[2 * r], edges[2 * r + 1]
        for p in range(NPASS):
            rid = 2 * p + core
            lo = rid * RNG

            zbase = sub * (ACC // NS)
            for k in range(6):
                pltpu.sync_copy(zrow,
                                acc_sp.at[pl.ds(zbase + 64 * k, 64)])
            pltpu.sync_copy(zrow.at[pl.ds(0, ACC // NS - 384)],
                            acc_sp.at[pl.ds(zbase + 384, ACC // NS - 384)])
            if with_counts:
                pltpu.sync_copy(zcnt.at[pl.ds(0, ACC // NS)],
                                cnt_sp.at[pl.ds(zbase, ACC // NS)])

            plsc.subcore_barrier()

            def _chunk(c, cnt):
                pltpu.sync_copy(src_h.at[sub, pl.ds(c * 112, 112)], srcv)
                pltpu.sync_copy(dst_h.at[sub, pl.ds(c * 112, 112)], dstv)

                def _scan(q, cnt):
                    ss, ds_, ms, cums = [], [], [], []
                    for jj in range(4):
                        j = q * 4 + jj
                        s16 = srcv[j]
                        d16 = dstv[j]
                        m = (d16 >= lo) & (d16 < lo + RNG)
                        ss.append(s16)
                        ds_.append(d16)
                        ms.append(m)
                        cums.append(plsc.cumsum(m.astype(jnp.int32)))
                    for jj in range(4):
                        pos = cnt + cums[jj] - 1
                        plsc.store_scatter(cg, [pos], ss[jj], mask=ms[jj])
                        plsc.store_scatter(cs, [pos], ds_[jj] - lo,
                                           mask=ms[jj])
                        cnt = cnt + cums[jj][15]
                    return cnt

                return lax.fori_loop(0, 28, _scan, cnt)

            cnt = lax.fori_loop(0, 7, _chunk, jnp.int32(0))

            for k in range(8):
                tpos = cnt + 16 * k + iota16
                plsc.store_scatter(cg, [tpos], trash_g)
                plsc.store_scatter(cs, [tpos], trash_s)

            nb = (cnt + 127) >> 7

            def _drain(u):
                dummy = cs.at[pl.ds(0, 128)]
                pltpu.make_async_copy(rowbufs[u], acc_sp.at[dummy],
                                      ssems[u]).wait()
                if with_counts:
                    pltpu.make_async_copy(onesv, cnt_sp.at[dummy],
                                          ssems[u]).wait()

            def _half(h, _):
                for u in range(2):
                    b = h * 2 + u

                    @pl.when(b < nb)
                    def _():
                        @pl.when(b >= 2)
                        def _():
                            _drain(u)
                        gslice = cg.at[pl.ds(b * 128, 128)]
                        pltpu.async_copy(tab.at[gslice], rowbufs[u],
                                         gsems[u])
                for u in range(2):
                    b = h * 2 + u

                    @pl.when(b < nb)
                    def _():
                        gslice = cg.at[pl.ds(b * 128, 128)]
                        sslice = cs.at[pl.ds(b * 128, 128)]
                        pltpu.make_async_copy(tab.at[gslice], rowbufs[u],
                                              gsems[u]).wait()
                        pltpu.async_copy(rowbufs[u], acc_sp.at[sslice],
                                         ssems[u], add=True)
                        if with_counts:
                            pltpu.async_copy(onesv, cnt_sp.at[sslice],
                                             ssems[u], add=True)
                return 0

            lax.fori_loop(0, (nb + 1) >> 1, _half, 0)
            for u in range(2):
                @pl.when(nb > u)
                def _():
                    _drain(u)

            plsc.subcore_barrier()
            wbase = sub * (RNG // NS)
            pltpu.sync_copy(acc_sp.at[pl.ds(wbase, RNG // NS)],
                            aggs[r].at[pl.ds(lo + wbase, RNG // NS)])
            if with_counts:
                pltpu.sync_copy(
                    cnt_sp.at[pl.ds(pl.multiple_of(wbase, 8), RNG // NS)],
                    cntv)
                pltpu.sync_copy(cntv, cnts[r].at[rid * NS + sub])
            plsc.subcore_barrier()


def _make_agg(nrels, with_counts):
    outs = [jax.ShapeDtypeStruct((NP, D), jnp.float32) for _ in range(nrels)]
    if with_counts:
        outs += [jax.ShapeDtypeStruct((2 * NPASS * NS, RNG // NS),
                                      jnp.float32)
                 for _ in range(nrels)]
    mesh = plsc.VectorSubcoreMesh(core_axis_name="c", subcore_axis_name="s",
                                  num_cores=NC, num_subcores=NS)
    return pl.kernel(
        functools.partial(_agg_body, nrels, with_counts),
        out_type=tuple(outs),
        mesh=mesh,
        scratch_types=[
            pltpu.VMEM_SHARED((ACC, D), jnp.float32),
            pltpu.VMEM_SHARED((ACC,), jnp.float32),
            pltpu.VMEM((112, 16), jnp.int32),
            pltpu.VMEM((112, 16), jnp.int32),
            pltpu.VMEM((CAP,), jnp.int32),
            pltpu.VMEM((CAP,), jnp.int32),
            pltpu.VMEM((128, D), jnp.float32),
            pltpu.VMEM((128, D), jnp.float32),
            pltpu.VMEM((64, D), jnp.float32),
            pltpu.VMEM((128,), jnp.float32),
            pltpu.VMEM((800,), jnp.float32),
            pltpu.VMEM((RNG // NS,), jnp.float32),
            pltpu.SemaphoreType.DMA,
            pltpu.SemaphoreType.DMA,
            pltpu.SemaphoreType.DMA,
            pltpu.SemaphoreType.DMA,
        ],
        compiler_params=pltpu.CompilerParams(use_tc_tiling_on_sc=False,
                                             needs_layout_passes=False),
        name=f"sc_agg{nrels}",
    )


_agg_cw = _make_agg(2, True)
_agg_r = _make_agg(1, True)
_agg1 = _make_agg(1, False)


def _paper1_body(aggc, aggw, cc, cw, xp, wlc, wlw, wrp, bp, xp1):
    invc = 1.0 / jnp.maximum(cc[...], 1.0)
    invw = 1.0 / jnp.maximum(cw[...], 1.0)
    f32 = jnp.float32
    hp = (jnp.dot(aggc[...] * invc, wlc[...], preferred_element_type=f32)
          + jnp.dot(aggw[...] * invw, wlw[...], preferred_element_type=f32)
          + jnp.dot(xp[...], wrp[...], preferred_element_type=f32) + bp[...])
    xp1[...] = jnp.maximum(hp, 0.0)


def _author1_body(aggr, cr, xa, wlr, wra, ba, xa1):
    invr = 1.0 / jnp.maximum(cr[...], 1.0)
    f32 = jnp.float32
    ha = (jnp.dot(aggr[...] * invr, wlr[...], preferred_element_type=f32)
          + jnp.dot(xa[...], wra[...], preferred_element_type=f32) + ba[...])
    xa1[...] = jnp.maximum(ha, 0.0)


def _layer2_body(i, aggc, aggw, cc, cw, xp1, bat, wlc, wlw, wrp, bp,
                 wlin, blin, pooled, cntb, final):
    invc = 1.0 / jnp.maximum(cc[...], 1.0)
    invw = 1.0 / jnp.maximum(cw[...], 1.0)
    f32 = jnp.float32
    hp = (jnp.dot(aggc[...] * invc, wlc[...], preferred_element_type=f32)
          + jnp.dot(aggw[...] * invw, wlw[...], preferred_element_type=f32)
          + jnp.dot(xp1[...], wrp[...], preferred_element_type=f32) + bp[...])
    xp2 = jnp.maximum(hp, 0.0)
    xp2 = jnp.where(xp2 < jnp.float32(1e30), xp2, jnp.float32(0.0))
    bb = bat[0]
    oh = (lax.broadcasted_iota(jnp.int32, (G, RB), 0)
          == jnp.broadcast_to(bb, (G, RB))).astype(f32)

    @pl.when(i == 0)
    def _():
        pooled[...] = jnp.zeros((G, D), f32)
        cntb[...] = jnp.zeros((G, D), f32)

    pooled[...] += jnp.dot(oh, xp2, preferred_element_type=f32)
    cntb[...] += jnp.broadcast_to(jnp.sum(oh, axis=1, keepdims=True), (G, D))

    @pl.when(i == GB - 1)
    def _():
        inv = 1.0 / jnp.maximum(cntb[...], 1.0)
        final[...] = (jnp.dot(pooled[...] * inv, wlin[...],
                              preferred_element_type=f32) + blin[...])


def _l2_with_i(*args):
    _layer2_body(pl.program_id(0), *args)


_row = pl.BlockSpec((RB, D), lambda i: (i, 0))
_col1 = pl.BlockSpec((RB, 1), lambda i: (i, 0))
_wfull = pl.BlockSpec((D, D), lambda i: (0, 0))
_bfull = pl.BlockSpec((1, D), lambda i: (0, 0))

_paper1 = pl.pallas_call(
    _paper1_body,
    grid=(GB,),
    in_specs=[_row, _row, _col1, _col1, _row, _wfull, _wfull, _wfull,
              _bfull],
    out_specs=_row,
    out_shape=jax.ShapeDtypeStruct((NP, D), jnp.float32),
    compiler_params=pltpu.CompilerParams(
        dimension_semantics=("arbitrary",)),
)

_author1 = pl.pallas_call(
    _author1_body,
    grid=(GB,),
    in_specs=[_row, _col1, _row, _wfull, _wfull, _bfull],
    out_specs=_row,
    out_shape=jax.ShapeDtypeStruct((NP, D), jnp.float32),
    compiler_params=pltpu.CompilerParams(
        dimension_semantics=("arbitrary",)),
)

_layer2 = pl.pallas_call(
    _l2_with_i,
    grid=(GB,),
    in_specs=[_row, _row, _col1, _col1, _row,
              pl.BlockSpec((1, 1, RB), lambda i: (i, 0, 0)),
              _wfull, _wfull, _wfull, _bfull,
              pl.BlockSpec((D, C), lambda i: (0, 0)),
              pl.BlockSpec((1, C), lambda i: (0, 0))],
    out_specs=[pl.BlockSpec((G, D), lambda i: (0, 0)),
               pl.BlockSpec((G, D), lambda i: (0, 0)),
               pl.BlockSpec((G, C), lambda i: (0, 0))],
    out_shape=[jax.ShapeDtypeStruct((G, D), jnp.float32),
               jax.ShapeDtypeStruct((G, D), jnp.float32),
               jax.ShapeDtypeStruct((G, C), jnp.float32)],
    compiler_params=pltpu.CompilerParams(
        dimension_semantics=("arbitrary",)),
)


def _pad_edges(ei):
    src = jnp.concatenate([ei[0], jnp.zeros((EP - E,), jnp.int32)])
    dst = jnp.concatenate([ei[1], jnp.full((EP - E,), 1 << 28, jnp.int32)])
    return src.reshape(NS, EPT // 16, 16), dst.reshape(NS, EPT // 16, 16)


def kernel(x_paper, x_author, edge_index_cites, edge_index_writes,
           edge_index_rev, batch, Wl1c, bl1c, Wr1c, Wl1w, bl1w, Wr1w,
           Wl1r, bl1r, Wr1r, Wl2c, bl2c, Wr2c, Wl2w, bl2w, Wr2w,
           Wl2r, bl2r, Wr2r, Wlin, blin):
    xp = x_paper
    xa = x_author
    sc_, dc_ = _pad_edges(edge_index_cites)
    sw_, dw_ = _pad_edges(edge_index_writes)
    sr_, dr_ = _pad_edges(edge_index_rev)
    bat = jnp.concatenate([batch, jnp.full((NP - N,), G, jnp.int32)])
    bat = bat.reshape(GB, 1, RB)

    aggc, aggw, cc, cw = _agg_cw(xp, xa, sc_, dc_, sw_, dw_)
    aggr, cr = _agg_r(xp, sr_, dr_)
    cc = cc.reshape(NP, 1)
    cw = cw.reshape(NP, 1)
    cr = cr.reshape(NP, 1)

    xp1 = _paper1(aggc, aggw, cc, cw, xp, Wl1c, Wl1w, (Wr1c + Wr1w),
                  (bl1c + bl1w).reshape(1, D))
    xa1 = _author1(aggr, cr, xa, Wl1r, Wr1r, bl1r.reshape(1, D))

    (aggc2,) = _agg1(xp1, sc_, dc_)
    (aggw2,) = _agg1(xa1, sw_, dw_)

    _, _, final = _layer2(aggc2, aggw2, cc, cw, xp1, bat,
                          Wl2c, Wl2w, (Wr2c + Wr2w),
                          (bl2c + bl2w).reshape(1, D),
                          Wlin, blin.reshape(1, C))
    return final


# speedup vs baseline: 5.2212x; 1.1927x over previous
import functools

import jax
import jax.numpy as jnp
from jax import lax
from jax.experimental import pallas as pl
from jax.experimental.pallas import tpu as pltpu
from jax.experimental.pallas import tpu_sc as plsc

N = 50000
E = 200000
D = 128
G = 64
C = 32

NC = 2
NS = 16
NPASS = 4
RNG = 6272
NP = 8 * RNG
ACC = RNG + 256
EPT = 12544
EP = NS * EPT
CAP = 12800
RB = 1792
GB = NP // RB


def _agg_body(nrels, with_counts, *refs):
    pos = 0
    tabs = refs[pos:pos + nrels]; pos += nrels
    edges = refs[pos:pos + 2 * nrels]; pos += 2 * nrels
    aggs = refs[pos:pos + nrels]; pos += nrels
    cnts = refs[pos:pos + nrels] if with_counts else ()
    if with_counts:
        pos += nrels
    (acc_sp, cnt_sp, srcv0, dstv0, srcv1, dstv1, cg, cs, rowbuf0, rowbuf1,
     zrow, onesv, zcnt, cntv, gsem0, gsem1, ssem0, ssem1, esem0,
     esem1) = refs[pos:]
    rowbufs = (rowbuf0, rowbuf1)
    gsems = (gsem0, gsem1)
    ssems = (ssem0, ssem1)
    srcvs = (srcv0, srcv1)
    dstvs = (dstv0, dstv1)
    esems = (esem0, esem1)

    core = lax.axis_index("c")
    sub = lax.axis_index("s")
    zero16 = jnp.zeros((16,), jnp.float32)
    one16 = jnp.ones((16,), jnp.float32)
    iota16 = lax.iota(jnp.int32, 16)

    @pl.loop(0, ACC // NS // 16 + 1)
    def _(k):
        zcnt[pl.ds(k * 16, 16)] = zero16

    @pl.loop(0, 8)
    def _(k):
        onesv[pl.ds(k * 16, 16)] = one16

    @pl.loop(0, 512)
    def _(k):
        zrow[k >> 3, pl.ds((k & 7) * 16, 16)] = zero16

    trash_s = RNG + ((sub * 16 + iota16) & 255)
    trash_g = sub * 16 + iota16

    for r in range(nrels):
        tab = tabs[r]
        src_h, dst_h = edges[2 * r], edges[2 * r + 1]

        def _pass(p, _, tab=tab, src_h=src_h, dst_h=dst_h, r=r):
            rid = 2 * p + core
            lo = rid * RNG

            zbase = sub * (ACC // NS)
            for k in range(6):
                pltpu.sync_copy(zrow,
                                acc_sp.at[pl.ds(zbase + 64 * k, 64)])
            pltpu.sync_copy(zrow.at[pl.ds(0, ACC // NS - 384)],
                            acc_sp.at[pl.ds(zbase + 384, ACC // NS - 384)])
            if with_counts:
                pltpu.sync_copy(zcnt.at[pl.ds(0, ACC // NS)],
                                cnt_sp.at[pl.ds(zbase, ACC // NS)])

            plsc.subcore_barrier()

            def _fire_edges(c, u):
                pltpu.async_copy(src_h.at[sub, pl.ds(c * 112, 112)],
                                 srcvs[u], esems[u])
                pltpu.async_copy(dst_h.at[sub, pl.ds(c * 112, 112)],
                                 dstvs[u], esems[u])

            def _wait_edges(c, u):
                pltpu.make_async_copy(src_h.at[sub, pl.ds(c * 112, 112)],
                                      srcvs[u], esems[u]).wait()
                pltpu.make_async_copy(dst_h.at[sub, pl.ds(c * 112, 112)],
                                      dstvs[u], esems[u]).wait()

            _fire_edges(0, 0)
            cnt = jnp.int32(0)
            for c in range(7):
                u = c & 1
                _wait_edges(c, u)
                if c + 1 < 7:
                    _fire_edges(c + 1, 1 - u)
                srcv = srcvs[u]
                dstv = dstvs[u]

                def _scan(q, cnt, srcv=srcv, dstv=dstv):
                    ss, ds_, ms, cums = [], [], [], []
                    for jj in range(8):
                        j = q * 8 + jj
                        s16 = srcv[j]
                        d16 = dstv[j]
                        m = (d16 >= lo) & (d16 < lo + RNG)
                        ss.append(s16)
                        ds_.append(d16)
                        ms.append(m)
                        cums.append(plsc.cumsum(m.astype(jnp.int32)))
                    for jj in range(8):
                        pos = cnt + cums[jj] - 1
                        plsc.store_scatter(cg, [pos], ss[jj], mask=ms[jj])
                        plsc.store_scatter(cs, [pos], ds_[jj] - lo,
                                           mask=ms[jj])
                        cnt = cnt + cums[jj][15]
                    return cnt

                cnt = lax.fori_loop(0, 14, _scan, cnt)

            for k in range(8):
                tpos = cnt + 16 * k + iota16
                plsc.store_scatter(cg, [tpos], trash_g)
                plsc.store_scatter(cs, [tpos], trash_s)

            nb = (cnt + 127) >> 7

            def _drain(u):
                dummy = cs.at[pl.ds(0, 128)]
                pltpu.make_async_copy(rowbufs[u], acc_sp.at[dummy],
                                      ssems[u]).wait()
                if with_counts:
                    pltpu.make_async_copy(onesv, cnt_sp.at[dummy],
                                          ssems[u]).wait()

            def _half(h, _):
                for u in range(2):
                    b = h * 2 + u

                    @pl.when(b < nb)
                    def _():
                        @pl.when(b >= 2)
                        def _():
                            _drain(u)
                        gslice = cg.at[pl.ds(b * 128, 128)]
                        pltpu.async_copy(tab.at[gslice], rowbufs[u],
                                         gsems[u])
                for u in range(2):
                    b = h * 2 + u

                    @pl.when(b < nb)
                    def _():
                        gslice = cg.at[pl.ds(b * 128, 128)]
                        sslice = cs.at[pl.ds(b * 128, 128)]
                        pltpu.make_async_copy(tab.at[gslice], rowbufs[u],
                                              gsems[u]).wait()
                        pltpu.async_copy(rowbufs[u], acc_sp.at[sslice],
                                         ssems[u], add=True)
                        if with_counts:
                            pltpu.async_copy(onesv, cnt_sp.at[sslice],
                                             ssems[u], add=True)
                return 0

            lax.fori_loop(0, (nb + 1) >> 1, _half, 0)
            for u in range(2):
                @pl.when(nb > u)
                def _():
                    _drain(u)

            plsc.subcore_barrier()
            wbase = sub * (RNG // NS)
            pltpu.sync_copy(acc_sp.at[pl.ds(wbase, RNG // NS)],
                            aggs[r].at[pl.ds(lo + wbase, RNG // NS)])
            if with_counts:
                pltpu.sync_copy(
                    cnt_sp.at[pl.ds(pl.multiple_of(wbase, 8), RNG // NS)],
                    cntv)
                pltpu.sync_copy(cntv, cnts[r].at[rid * NS + sub])
            plsc.subcore_barrier()
            return 0

        lax.fori_loop(0, NPASS, _pass, 0)


def _make_agg(nrels, with_counts):
    outs = [jax.ShapeDtypeStruct((NP, D), jnp.float32) for _ in range(nrels)]
    if with_counts:
        outs += [jax.ShapeDtypeStruct((2 * NPASS * NS, RNG // NS),
                                      jnp.float32)
                 for _ in range(nrels)]
    mesh = plsc.VectorSubcoreMesh(core_axis_name="c", subcore_axis_name="s",
                                  num_cores=NC, num_subcores=NS)
    return pl.kernel(
        functools.partial(_agg_body, nrels, with_counts),
        out_type=tuple(outs),
        mesh=mesh,
        scratch_types=[
            pltpu.VMEM_SHARED((ACC, D), jnp.float32),
            pltpu.VMEM_SHARED((ACC,), jnp.float32),
            pltpu.VMEM((112, 16), jnp.int32),
            pltpu.VMEM((112, 16), jnp.int32),
            pltpu.VMEM((112, 16), jnp.int32),
            pltpu.VMEM((112, 16), jnp.int32),
            pltpu.VMEM((CAP,), jnp.int32),
            pltpu.VMEM((CAP,), jnp.int32),
            pltpu.VMEM((128, D), jnp.float32),
            pltpu.VMEM((128, D), jnp.float32),
            pltpu.VMEM((64, D), jnp.float32),
            pltpu.VMEM((128,), jnp.float32),
            pltpu.VMEM((800,), jnp.float32),
            pltpu.VMEM((RNG // NS,), jnp.float32),
            pltpu.SemaphoreType.DMA,
            pltpu.SemaphoreType.DMA,
            pltpu.SemaphoreType.DMA,
            pltpu.SemaphoreType.DMA,
            pltpu.SemaphoreType.DMA,
            pltpu.SemaphoreType.DMA,
        ],
        compiler_params=pltpu.CompilerParams(use_tc_tiling_on_sc=False,
                                             needs_layout_passes=False),
        name=f"sc_agg{nrels}",
    )


_agg_cw = _make_agg(2, True)
_agg_r = _make_agg(1, True)
_agg1 = _make_agg(1, False)


def _paper1_body(aggc, aggw, cc, cw, xp, wlc, wlw, wrp, bp, xp1):
    invc = 1.0 / jnp.maximum(cc[...], 1.0)
    invw = 1.0 / jnp.maximum(cw[...], 1.0)
    f32 = jnp.float32
    hp = (jnp.dot(aggc[...] * invc, wlc[...], preferred_element_type=f32)
          + jnp.dot(aggw[...] * invw, wlw[...], preferred_element_type=f32)
          + jnp.dot(xp[...], wrp[...], preferred_element_type=f32) + bp[...])
    xp1[...] = jnp.maximum(hp, 0.0)


def _author1_body(aggr, cr, xa, wlr, wra, ba, xa1):
    invr = 1.0 / jnp.maximum(cr[...], 1.0)
    f32 = jnp.float32
    ha = (jnp.dot(aggr[...] * invr, wlr[...], preferred_element_type=f32)
          + jnp.dot(xa[...], wra[...], preferred_element_type=f32) + ba[...])
    xa1[...] = jnp.maximum(ha, 0.0)


def _layer2_body(i, aggc, aggw, cc, cw, xp1, bat, wlc, wlw, wrp, bp,
                 wlin, blin, pooled, cntb, final):
    invc = 1.0 / jnp.maximum(cc[...], 1.0)
    invw = 1.0 / jnp.maximum(cw[...], 1.0)
    f32 = jnp.float32
    hp = (jnp.dot(aggc[...] * invc, wlc[...], preferred_element_type=f32)
          + jnp.dot(aggw[...] * invw, wlw[...], preferred_element_type=f32)
          + jnp.dot(xp1[...], wrp[...], preferred_element_type=f32) + bp[...])
    xp2 = jnp.maximum(hp, 0.0)
    xp2 = jnp.where(xp2 < jnp.float32(1e30), xp2, jnp.float32(0.0))
    bb = bat[0]
    oh = (lax.broadcasted_iota(jnp.int32, (G, RB), 0)
          == jnp.broadcast_to(bb, (G, RB))).astype(f32)

    @pl.when(i == 0)
    def _():
        pooled[...] = jnp.zeros((G, D), f32)
        cntb[...] = jnp.zeros((G, D), f32)

    pooled[...] += jnp.dot(oh, xp2, preferred_element_type=f32)
    cntb[...] += jnp.broadcast_to(jnp.sum(oh, axis=1, keepdims=True), (G, D))

    @pl.when(i == GB - 1)
    def _():
        inv = 1.0 / jnp.maximum(cntb[...], 1.0)
        final[...] = (jnp.dot(pooled[...] * inv, wlin[...],
                              preferred_element_type=f32) + blin[...])


def _l2_with_i(*args):
    _layer2_body(pl.program_id(0), *args)


_row = pl.BlockSpec((RB, D), lambda i: (i, 0))
_col1 = pl.BlockSpec((RB, 1), lambda i: (i, 0))
_wfull = pl.BlockSpec((D, D), lambda i: (0, 0))
_bfull = pl.BlockSpec((1, D), lambda i: (0, 0))

_paper1 = pl.pallas_call(
    _paper1_body,
    grid=(GB,),
    in_specs=[_row, _row, _col1, _col1, _row, _wfull, _wfull, _wfull,
              _bfull],
    out_specs=_row,
    out_shape=jax.ShapeDtypeStruct((NP, D), jnp.float32),
    compiler_params=pltpu.CompilerParams(
        dimension_semantics=("arbitrary",)),
)

_author1 = pl.pallas_call(
    _author1_body,
    grid=(GB,),
    in_specs=[_row, _col1, _row, _wfull, _wfull, _bfull],
    out_specs=_row,
    out_shape=jax.ShapeDtypeStruct((NP, D), jnp.float32),
    compiler_params=pltpu.CompilerParams(
        dimension_semantics=("arbitrary",)),
)

_layer2 = pl.pallas_call(
    _l2_with_i,
    grid=(GB,),
    in_specs=[_row, _row, _col1, _col1, _row,
              pl.BlockSpec((1, 1, RB), lambda i: (i, 0, 0)),
              _wfull, _wfull, _wfull, _bfull,
              pl.BlockSpec((D, C), lambda i: (0, 0)),
              pl.BlockSpec((1, C), lambda i: (0, 0))],
    out_specs=[pl.BlockSpec((G, D), lambda i: (0, 0)),
               pl.BlockSpec((G, D), lambda i: (0, 0)),
               pl.BlockSpec((G, C), lambda i: (0, 0))],
    out_shape=[jax.ShapeDtypeStruct((G, D), jnp.float32),
               jax.ShapeDtypeStruct((G, D), jnp.float32),
               jax.ShapeDtypeStruct((G, C), jnp.float32)],
    compiler_params=pltpu.CompilerParams(
        dimension_semantics=("arbitrary",)),
)


def _pad_edges(ei):
    src = jnp.concatenate([ei[0], jnp.zeros((EP - E,), jnp.int32)])
    dst = jnp.concatenate([ei[1], jnp.full((EP - E,), 1 << 28, jnp.int32)])
    return src.reshape(NS, EPT // 16, 16), dst.reshape(NS, EPT // 16, 16)


def kernel(x_paper, x_author, edge_index_cites, edge_index_writes,
           edge_index_rev, batch, Wl1c, bl1c, Wr1c, Wl1w, bl1w, Wr1w,
           Wl1r, bl1r, Wr1r, Wl2c, bl2c, Wr2c, Wl2w, bl2w, Wr2w,
           Wl2r, bl2r, Wr2r, Wlin, blin):
    xp = x_paper
    xa = x_author
    sc_, dc_ = _pad_edges(edge_index_cites)
    sw_, dw_ = _pad_edges(edge_index_writes)
    sr_, dr_ = _pad_edges(edge_index_rev)
    bat = jnp.concatenate([batch, jnp.full((NP - N,), G, jnp.int32)])
    bat = bat.reshape(GB, 1, RB)

    aggc, aggw, cc, cw = _agg_cw(xp, xa, sc_, dc_, sw_, dw_)
    aggr, cr = _agg_r(xp, sr_, dr_)
    cc = cc.reshape(NP, 1)
    cw = cw.reshape(NP, 1)
    cr = cr.reshape(NP, 1)

    xp1 = _paper1(aggc, aggw, cc, cw, xp, Wl1c, Wl1w, (Wr1c + Wr1w),
                  (bl1c + bl1w).reshape(1, D))
    xa1 = _author1(aggr, cr, xa, Wl1r, Wr1r, bl1r.reshape(1, D))

    (aggc2,) = _agg1(xp1, sc_, dc_)
    (aggw2,) = _agg1(xa1, sw_, dw_)

    _, _, final = _layer2(aggc2, aggw2, cc, cw, xp1, bat,
                          Wl2c, Wl2w, (Wr2c + Wr2w),
                          (bl2c + bl2w).reshape(1, D),
                          Wlin, blin.reshape(1, C))
    return final


# speedup vs baseline: 5.9799x; 1.1453x over previous
import functools

import jax
import jax.numpy as jnp
from jax import lax
from jax.experimental import pallas as pl
from jax.experimental.pallas import tpu as pltpu
from jax.experimental.pallas import tpu_sc as plsc

N = 50000
E = 200000
D = 128
G = 64
C = 32

NC = 2
NS = 16
NPASS = 4
RNG = 6272
NP = 8 * RNG
ACC = RNG + 256
EPT = 12544
EP = NS * EPT
CAP = 12800
RB = 1792
GB = NP // RB


def _agg_body(nrels, with_counts, *refs):
    pos = 0
    tabs = refs[pos:pos + nrels]; pos += nrels
    edges = refs[pos:pos + 2 * nrels]; pos += 2 * nrels
    aggs = refs[pos:pos + nrels]; pos += nrels
    cnts = refs[pos:pos + nrels] if with_counts else ()
    if with_counts:
        pos += nrels
    (acc_sp, cnt_sp, srcv0, dstv0, srcv1, dstv1, cg, cs, rowbuf0, rowbuf1,
     rowbuf2, rowbuf3, zrow, onesv, zcnt, cntv, gsem0, gsem1, gsem2,
     gsem3, ssem0, ssem1, ssem2, ssem3, esem0, esem1) = refs[pos:]
    rowbufs = (rowbuf0, rowbuf1, rowbuf2, rowbuf3)
    gsems = (gsem0, gsem1, gsem2, gsem3)
    ssems = (ssem0, ssem1, ssem2, ssem3)
    srcvs = (srcv0, srcv1)
    dstvs = (dstv0, dstv1)
    esems = (esem0, esem1)

    core = lax.axis_index("c")
    sub = lax.axis_index("s")
    zero16 = jnp.zeros((16,), jnp.float32)
    one16 = jnp.ones((16,), jnp.float32)
    iota16 = lax.iota(jnp.int32, 16)

    @pl.loop(0, ACC // NS // 16 + 1)
    def _(k):
        zcnt[pl.ds(k * 16, 16)] = zero16

    @pl.loop(0, 4)
    def _(k):
        onesv[pl.ds(k * 16, 16)] = one16

    @pl.loop(0, 512)
    def _(k):
        zrow[k >> 3, pl.ds((k & 7) * 16, 16)] = zero16

    trash_s = RNG + ((sub * 16 + iota16) & 255)
    trash_g = sub * 16 + iota16

    for r in range(nrels):
        tab = tabs[r]
        src_h, dst_h = edges[2 * r], edges[2 * r + 1]

        def _pass(p, _, tab=tab, src_h=src_h, dst_h=dst_h, r=r):
            rid = 2 * p + core
            lo = rid * RNG

            zbase = sub * (ACC // NS)
            for k in range(6):
                pltpu.sync_copy(zrow,
                                acc_sp.at[pl.ds(zbase + 64 * k, 64)])
            pltpu.sync_copy(zrow.at[pl.ds(0, ACC // NS - 384)],
                            acc_sp.at[pl.ds(zbase + 384, ACC // NS - 384)])
            if with_counts:
                pltpu.sync_copy(zcnt.at[pl.ds(0, ACC // NS)],
                                cnt_sp.at[pl.ds(zbase, ACC // NS)])

            plsc.subcore_barrier()

            def _fire_edges(c, u):
                pltpu.async_copy(src_h.at[sub, pl.ds(c * 112, 112)],
                                 srcvs[u], esems[u])
                pltpu.async_copy(dst_h.at[sub, pl.ds(c * 112, 112)],
                                 dstvs[u], esems[u])

            def _wait_edges(c, u):
                pltpu.make_async_copy(src_h.at[sub, pl.ds(c * 112, 112)],
                                      srcvs[u], esems[u]).wait()
                pltpu.make_async_copy(dst_h.at[sub, pl.ds(c * 112, 112)],
                                      dstvs[u], esems[u]).wait()

            _fire_edges(0, 0)
            cnt = jnp.int32(0)
            for c in range(7):
                u = c & 1
                _wait_edges(c, u)
                if c + 1 < 7:
                    _fire_edges(c + 1, 1 - u)
                srcv = srcvs[u]
                dstv = dstvs[u]

                def _scan(q, cnt, srcv=srcv, dstv=dstv):
                    ss, ds_, ms, cums = [], [], [], []
                    for jj in range(8):
                        j = q * 8 + jj
                        s16 = srcv[j]
                        d16 = dstv[j]
                        m = (d16 >= lo) & (d16 < lo + RNG)
                        ss.append(s16)
                        ds_.append(d16)
                        ms.append(m)
                        cums.append(plsc.cumsum(m.astype(jnp.int32)))
                    for jj in range(8):
                        pos = cnt + cums[jj] - 1
                        plsc.store_scatter(cg, [pos], ss[jj], mask=ms[jj])
                        plsc.store_scatter(cs, [pos], ds_[jj] - lo,
                                           mask=ms[jj])
                        cnt = cnt + cums[jj][15]
                    return cnt

                cnt = lax.fori_loop(0, 14, _scan, cnt)

            for k in range(4):
                tpos = cnt + 16 * k + iota16
                plsc.store_scatter(cg, [tpos], trash_g)
                plsc.store_scatter(cs, [tpos], trash_s)

            nb = (cnt + 63) >> 6

            def _drain(u):
                dummy = cs.at[pl.ds(0, 64)]
                pltpu.make_async_copy(rowbufs[u], acc_sp.at[dummy],
                                      ssems[u]).wait()
                if with_counts:
                    pltpu.make_async_copy(onesv, cnt_sp.at[dummy],
                                          ssems[u]).wait()

            def _quad(h, _):
                for u in range(4):
                    b = h * 4 + u

                    @pl.when(b < nb)
                    def _():
                        @pl.when(b >= 4)
                        def _():
                            _drain(u)
                        gslice = cg.at[pl.ds(b * 64, 64)]
                        pltpu.async_copy(tab.at[gslice], rowbufs[u],
                                         gsems[u])
                for u in range(4):
                    b = h * 4 + u

                    @pl.when(b < nb)
                    def _():
                        gslice = cg.at[pl.ds(b * 64, 64)]
                        sslice = cs.at[pl.ds(b * 64, 64)]
                        pltpu.make_async_copy(tab.at[gslice], rowbufs[u],
                                              gsems[u]).wait()
                        pltpu.async_copy(rowbufs[u], acc_sp.at[sslice],
                                         ssems[u], add=True)
                        if with_counts:
                            pltpu.async_copy(onesv, cnt_sp.at[sslice],
                                             ssems[u], add=True)
                return 0

            lax.fori_loop(0, (nb + 3) >> 2, _quad, 0)
            for u in range(4):
                @pl.when(nb > u)
                def _():
                    _drain(u)

            plsc.subcore_barrier()
            wbase = sub * (RNG // NS)
            pltpu.sync_copy(acc_sp.at[pl.ds(wbase, RNG // NS)],
                            aggs[r].at[pl.ds(lo + wbase, RNG // NS)])
            if with_counts:
                pltpu.sync_copy(
                    cnt_sp.at[pl.ds(pl.multiple_of(wbase, 8), RNG // NS)],
                    cntv)
                pltpu.sync_copy(cntv, cnts[r].at[rid * NS + sub])
            plsc.subcore_barrier()
            return 0

        lax.fori_loop(0, NPASS, _pass, 0)


def _make_agg(nrels, with_counts):
    outs = [jax.ShapeDtypeStruct((NP, D), jnp.float32) for _ in range(nrels)]
    if with_counts:
        outs += [jax.ShapeDtypeStruct((2 * NPASS * NS, RNG // NS),
                                      jnp.float32)
                 for _ in range(nrels)]
    mesh = plsc.VectorSubcoreMesh(core_axis_name="c", subcore_axis_name="s",
                                  num_cores=NC, num_subcores=NS)
    return pl.kernel(
        functools.partial(_agg_body, nrels, with_counts),
        out_type=tuple(outs),
        mesh=mesh,
        scratch_types=[
            pltpu.VMEM_SHARED((ACC, D), jnp.float32),
            pltpu.VMEM_SHARED((ACC,), jnp.float32),
            pltpu.VMEM((112, 16), jnp.int32),
            pltpu.VMEM((112, 16), jnp.int32),
            pltpu.VMEM((112, 16), jnp.int32),
            pltpu.VMEM((112, 16), jnp.int32),
            pltpu.VMEM((CAP,), jnp.int32),
            pltpu.VMEM((CAP,), jnp.int32),
            pltpu.VMEM((64, D), jnp.float32),
            pltpu.VMEM((64, D), jnp.float32),
            pltpu.VMEM((64, D), jnp.float32),
            pltpu.VMEM((64, D), jnp.float32),
            pltpu.VMEM((64, D), jnp.float32),
            pltpu.VMEM((64,), jnp.float32),
            pltpu.VMEM((800,), jnp.float32),
            pltpu.VMEM((RNG // NS,), jnp.float32),
            pltpu.SemaphoreType.DMA,
            pltpu.SemaphoreType.DMA,
            pltpu.SemaphoreType.DMA,
            pltpu.SemaphoreType.DMA,
            pltpu.SemaphoreType.DMA,
            pltpu.SemaphoreType.DMA,
            pltpu.SemaphoreType.DMA,
            pltpu.SemaphoreType.DMA,
            pltpu.SemaphoreType.DMA,
            pltpu.SemaphoreType.DMA,
        ],
        compiler_params=pltpu.CompilerParams(use_tc_tiling_on_sc=False,
                                             needs_layout_passes=False),
        name=f"sc_agg{nrels}",
    )


_agg_cw = _make_agg(2, True)
_agg_r = _make_agg(1, True)
_agg1 = _make_agg(1, False)


def _paper1_body(aggc, aggw, cc, cw, xp, wlc, wlw, wrp, bp, xp1):
    invc = 1.0 / jnp.maximum(cc[...], 1.0)
    invw = 1.0 / jnp.maximum(cw[...], 1.0)
    f32 = jnp.float32
    hp = (jnp.dot(aggc[...] * invc, wlc[...], preferred_element_type=f32)
          + jnp.dot(aggw[...] * invw, wlw[...], preferred_element_type=f32)
          + jnp.dot(xp[...], wrp[...], preferred_element_type=f32) + bp[...])
    xp1[...] = jnp.maximum(hp, 0.0)


def _author1_body(aggr, cr, xa, wlr, wra, ba, xa1):
    invr = 1.0 / jnp.maximum(cr[...], 1.0)
    f32 = jnp.float32
    ha = (jnp.dot(aggr[...] * invr, wlr[...], preferred_element_type=f32)
          + jnp.dot(xa[...], wra[...], preferred_element_type=f32) + ba[...])
    xa1[...] = jnp.maximum(ha, 0.0)


def _layer2_body(i, aggc, aggw, cc, cw, xp1, bat, wlc, wlw, wrp, bp,
                 wlin, blin, pooled, cntb, final):
    invc = 1.0 / jnp.maximum(cc[...], 1.0)
    invw = 1.0 / jnp.maximum(cw[...], 1.0)
    f32 = jnp.float32
    hp = (jnp.dot(aggc[...] * invc, wlc[...], preferred_element_type=f32)
          + jnp.dot(aggw[...] * invw, wlw[...], preferred_element_type=f32)
          + jnp.dot(xp1[...], wrp[...], preferred_element_type=f32) + bp[...])
    xp2 = jnp.maximum(hp, 0.0)
    xp2 = jnp.where(xp2 < jnp.float32(1e30), xp2, jnp.float32(0.0))
    bb = bat[0]
    oh = (lax.broadcasted_iota(jnp.int32, (G, RB), 0)
          == jnp.broadcast_to(bb, (G, RB))).astype(f32)

    @pl.when(i == 0)
    def _():
        pooled[...] = jnp.zeros((G, D), f32)
        cntb[...] = jnp.zeros((G, D), f32)

    pooled[...] += jnp.dot(oh, xp2, preferred_element_type=f32)
    cntb[...] += jnp.broadcast_to(jnp.sum(oh, axis=1, keepdims=True), (G, D))

    @pl.when(i == GB - 1)
    def _():
        inv = 1.0 / jnp.maximum(cntb[...], 1.0)
        final[...] = (jnp.dot(pooled[...] * inv, wlin[...],
                              preferred_element_type=f32) + blin[...])


def _l2_with_i(*args):
    _layer2_body(pl.program_id(0), *args)


_row = pl.BlockSpec((RB, D), lambda i: (i, 0))
_col1 = pl.BlockSpec((RB, 1), lambda i: (i, 0))
_wfull = pl.BlockSpec((D, D), lambda i: (0, 0))
_bfull = pl.BlockSpec((1, D), lambda i: (0, 0))

_paper1 = pl.pallas_call(
    _paper1_body,
    grid=(GB,),
    in_specs=[_row, _row, _col1, _col1, _row, _wfull, _wfull, _wfull,
              _bfull],
    out_specs=_row,
    out_shape=jax.ShapeDtypeStruct((NP, D), jnp.float32),
    compiler_params=pltpu.CompilerParams(
        dimension_semantics=("arbitrary",)),
)

_author1 = pl.pallas_call(
    _author1_body,
    grid=(GB,),
    in_specs=[_row, _col1, _row, _wfull, _wfull, _bfull],
    out_specs=_row,
    out_shape=jax.ShapeDtypeStruct((NP, D), jnp.float32),
    compiler_params=pltpu.CompilerParams(
        dimension_semantics=("arbitrary",)),
)

_layer2 = pl.pallas_call(
    _l2_with_i,
    grid=(GB,),
    in_specs=[_row, _row, _col1, _col1, _row,
              pl.BlockSpec((1, 1, RB), lambda i: (i, 0, 0)),
              _wfull, _wfull, _wfull, _bfull,
              pl.BlockSpec((D, C), lambda i: (0, 0)),
              pl.BlockSpec((1, C), lambda i: (0, 0))],
    out_specs=[pl.BlockSpec((G, D), lambda i: (0, 0)),
               pl.BlockSpec((G, D), lambda i: (0, 0)),
               pl.BlockSpec((G, C), lambda i: (0, 0))],
    out_shape=[jax.ShapeDtypeStruct((G, D), jnp.float32),
               jax.ShapeDtypeStruct((G, D), jnp.float32),
               jax.ShapeDtypeStruct((G, C), jnp.float32)],
    compiler_params=pltpu.CompilerParams(
        dimension_semantics=("arbitrary",)),
)


def _pad_edges(ei):
    src = jnp.concatenate([ei[0], jnp.zeros((EP - E,), jnp.int32)])
    dst = jnp.concatenate([ei[1], jnp.full((EP - E,), 1 << 28, jnp.int32)])
    return src.reshape(NS, EPT // 16, 16), dst.reshape(NS, EPT // 16, 16)


def kernel(x_paper, x_author, edge_index_cites, edge_index_writes,
           edge_index_rev, batch, Wl1c, bl1c, Wr1c, Wl1w, bl1w, Wr1w,
           Wl1r, bl1r, Wr1r, Wl2c, bl2c, Wr2c, Wl2w, bl2w, Wr2w,
           Wl2r, bl2r, Wr2r, Wlin, blin):
    xp = x_paper
    xa = x_author
    sc_, dc_ = _pad_edges(edge_index_cites)
    sw_, dw_ = _pad_edges(edge_index_writes)
    sr_, dr_ = _pad_edges(edge_index_rev)
    bat = jnp.concatenate([batch, jnp.full((NP - N,), G, jnp.int32)])
    bat = bat.reshape(GB, 1, RB)

    aggc, aggw, cc, cw = _agg_cw(xp, xa, sc_, dc_, sw_, dw_)
    aggr, cr = _agg_r(xp, sr_, dr_)
    cc = cc.reshape(NP, 1)
    cw = cw.reshape(NP, 1)
    cr = cr.reshape(NP, 1)

    xp1 = _paper1(aggc, aggw, cc, cw, xp, Wl1c, Wl1w, (Wr1c + Wr1w),
                  (bl1c + bl1w).reshape(1, D))
    xa1 = _author1(aggr, cr, xa, Wl1r, Wr1r, bl1r.reshape(1, D))

    (aggc2,) = _agg1(xp1, sc_, dc_)
    (aggw2,) = _agg1(xa1, sw_, dw_)

    _, _, final = _layer2(aggc2, aggw2, cc, cw, xp1, bat,
                          Wl2c, Wl2w, (Wr2c + Wr2w),
                          (bl2c + bl2w).reshape(1, D),
                          Wlin, blin.reshape(1, C))
    return final


# speedup vs baseline: 6.0965x; 1.0195x over previous
import functools

import jax
import jax.numpy as jnp
from jax import lax
from jax.experimental import pallas as pl
from jax.experimental.pallas import tpu as pltpu
from jax.experimental.pallas import tpu_sc as plsc

N = 50000
E = 200000
D = 128
G = 64
C = 32

NC = 2
NS = 16
NPASS = 4
RNG = 6272
NP = 8 * RNG
ACC = RNG + 256
EPT = 12544
EP = NS * EPT
CAP = 12800
RB = 1792
GB = NP // RB


def _agg_body(nrels, with_counts, *refs):
    pos = 0
    tabs = refs[pos:pos + nrels]; pos += nrels
    edges = refs[pos:pos + 2 * nrels]; pos += 2 * nrels
    aggs = refs[pos:pos + nrels]; pos += nrels
    cnts = refs[pos:pos + nrels] if with_counts else ()
    if with_counts:
        pos += nrels
    nscr = len(refs) - pos
    scr = refs[pos:]
    (acc_sp, cnt_sp, srcv0, dstv0, srcv1, dstv1, cg, cs) = scr[:8]
    rowbufs = scr[8:16]
    zrow, onesv, zcnt, cntv = scr[16:20]
    gsems = scr[20:28]
    ssems = scr[28:36]
    esems = scr[36:38]
    srcvs = (srcv0, srcv1)
    dstvs = (dstv0, dstv1)

    core = lax.axis_index("c")
    sub = lax.axis_index("s")
    zero16 = jnp.zeros((16,), jnp.float32)
    one16 = jnp.ones((16,), jnp.float32)
    iota16 = lax.iota(jnp.int32, 16)

    @pl.loop(0, ACC // NS // 16 + 1)
    def _(k):
        zcnt[pl.ds(k * 16, 16)] = zero16

    @pl.loop(0, 2)
    def _(k):
        onesv[pl.ds(k * 16, 16)] = one16

    @pl.loop(0, 512)
    def _(k):
        zrow[k >> 3, pl.ds((k & 7) * 16, 16)] = zero16

    trash_s = RNG + ((sub * 16 + iota16) & 255)
    trash_g = sub * 16 + iota16

    for r in range(nrels):
        tab = tabs[r]
        src_h, dst_h = edges[2 * r], edges[2 * r + 1]

        def _pass(p, _, tab=tab, src_h=src_h, dst_h=dst_h, r=r):
            rid = 2 * p + core
            lo = rid * RNG

            zbase = sub * (ACC // NS)
            for k in range(6):
                pltpu.sync_copy(zrow,
                                acc_sp.at[pl.ds(zbase + 64 * k, 64)])
            pltpu.sync_copy(zrow.at[pl.ds(0, ACC // NS - 384)],
                            acc_sp.at[pl.ds(zbase + 384, ACC // NS - 384)])
            if with_counts:
                pltpu.sync_copy(zcnt.at[pl.ds(0, ACC // NS)],
                                cnt_sp.at[pl.ds(zbase, ACC // NS)])

            plsc.subcore_barrier()

            def _fire_edges(c, u):
                pltpu.async_copy(src_h.at[sub, pl.ds(c * 112, 112)],
                                 srcvs[u], esems[u])
                pltpu.async_copy(dst_h.at[sub, pl.ds(c * 112, 112)],
                                 dstvs[u], esems[u])

            def _wait_edges(c, u):
                pltpu.make_async_copy(src_h.at[sub, pl.ds(c * 112, 112)],
                                      srcvs[u], esems[u]).wait()
                pltpu.make_async_copy(dst_h.at[sub, pl.ds(c * 112, 112)],
                                      dstvs[u], esems[u]).wait()

            _fire_edges(0, 0)
            cnt = jnp.int32(0)
            for c in range(7):
                u = c & 1
                _wait_edges(c, u)
                if c + 1 < 7:
                    _fire_edges(c + 1, 1 - u)
                srcv = srcvs[u]
                dstv = dstvs[u]

                def _scan(q, cnt, srcv=srcv, dstv=dstv):
                    ss, ds_, ms, cums = [], [], [], []
                    for jj in range(8):
                        j = q * 8 + jj
                        s16 = srcv[j]
                        d16 = dstv[j]
                        m = (d16 >= lo) & (d16 < lo + RNG)
                        ss.append(s16)
                        ds_.append(d16)
                        ms.append(m)
                        cums.append(plsc.cumsum(m.astype(jnp.int32)))
                    for jj in range(8):
                        pos = cnt + cums[jj] - 1
                        plsc.store_scatter(cg, [pos], ss[jj], mask=ms[jj])
                        plsc.store_scatter(cs, [pos], ds_[jj] - lo,
                                           mask=ms[jj])
                        cnt = cnt + cums[jj][15]
                    return cnt

                cnt = lax.fori_loop(0, 14, _scan, cnt)

            for k in range(4):
                tpos = cnt + 16 * k + iota16
                plsc.store_scatter(cg, [tpos], trash_g)
                plsc.store_scatter(cs, [tpos], trash_s)

            nb = (cnt + 31) >> 5

            def _drain(u):
                dummy = cs.at[pl.ds(0, 32)]
                pltpu.make_async_copy(rowbufs[u], acc_sp.at[dummy],
                                      ssems[u]).wait()
                if with_counts:
                    pltpu.make_async_copy(onesv, cnt_sp.at[dummy],
                                          ssems[u]).wait()

            def _oct(h, _):
                for u in range(8):
                    b = h * 8 + u

                    @pl.when(b < nb)
                    def _():
                        @pl.when(b >= 8)
                        def _():
                            _drain(u)
                        gslice = cg.at[pl.ds(b * 32, 32)]
                        pltpu.async_copy(tab.at[gslice], rowbufs[u],
                                         gsems[u])
                for u in range(8):
                    b = h * 8 + u

                    @pl.when(b < nb)
                    def _():
                        gslice = cg.at[pl.ds(b * 32, 32)]
                        sslice = cs.at[pl.ds(b * 32, 32)]
                        pltpu.make_async_copy(tab.at[gslice], rowbufs[u],
                                              gsems[u]).wait()
                        pltpu.async_copy(rowbufs[u], acc_sp.at[sslice],
                                         ssems[u], add=True)
                        if with_counts:
                            pltpu.async_copy(onesv, cnt_sp.at[sslice],
                                             ssems[u], add=True)
                return 0

            lax.fori_loop(0, (nb + 7) >> 3, _oct, 0)
            for u in range(8):
                @pl.when(nb > u)
                def _():
                    _drain(u)

            plsc.subcore_barrier()
            wbase = sub * (RNG // NS)
            pltpu.sync_copy(acc_sp.at[pl.ds(wbase, RNG // NS)],
                            aggs[r].at[pl.ds(lo + wbase, RNG // NS)])
            if with_counts:
                pltpu.sync_copy(
                    cnt_sp.at[pl.ds(pl.multiple_of(wbase, 8), RNG // NS)],
                    cntv)
                pltpu.sync_copy(cntv, cnts[r].at[rid * NS + sub])
            plsc.subcore_barrier()
            return 0

        lax.fori_loop(0, NPASS, _pass, 0)


def _make_agg(nrels, with_counts):
    outs = [jax.ShapeDtypeStruct((NP, D), jnp.float32) for _ in range(nrels)]
    if with_counts:
        outs += [jax.ShapeDtypeStruct((2 * NPASS * NS, RNG // NS),
                                      jnp.float32)
                 for _ in range(nrels)]
    mesh = plsc.VectorSubcoreMesh(core_axis_name="c", subcore_axis_name="s",
                                  num_cores=NC, num_subcores=NS)
    return pl.kernel(
        functools.partial(_agg_body, nrels, with_counts),
        out_type=tuple(outs),
        mesh=mesh,
        scratch_types=[
            pltpu.VMEM_SHARED((ACC, D), jnp.float32),
            pltpu.VMEM_SHARED((ACC,), jnp.float32),
            pltpu.VMEM((112, 16), jnp.int32),
            pltpu.VMEM((112, 16), jnp.int32),
            pltpu.VMEM((112, 16), jnp.int32),
            pltpu.VMEM((112, 16), jnp.int32),
            pltpu.VMEM((CAP,), jnp.int32),
            pltpu.VMEM((CAP,), jnp.int32),
        ] + [pltpu.VMEM((32, D), jnp.float32)] * 8 + [
            pltpu.VMEM((64, D), jnp.float32),
            pltpu.VMEM((32,), jnp.float32),
            pltpu.VMEM((800,), jnp.float32),
            pltpu.VMEM((RNG // NS,), jnp.float32),
        ] + [pltpu.SemaphoreType.DMA] * 18,
        compiler_params=pltpu.CompilerParams(use_tc_tiling_on_sc=False,
                                             needs_layout_passes=False),
        name=f"sc_agg{nrels}",
    )


_agg_cw = _make_agg(2, True)
_agg_r = _make_agg(1, True)
_agg1 = _make_agg(1, False)


def _paper1_body(aggc, aggw, cc, cw, xp, wlc, wlw, wrp, bp, xp1):
    invc = 1.0 / jnp.maximum(cc[...], 1.0)
    invw = 1.0 / jnp.maximum(cw[...], 1.0)
    f32 = jnp.float32
    hp = (jnp.dot(aggc[...] * invc, wlc[...], preferred_element_type=f32)
          + jnp.dot(aggw[...] * invw, wlw[...], preferred_element_type=f32)
          + jnp.dot(xp[...], wrp[...], preferred_element_type=f32) + bp[...])
    xp1[...] = jnp.maximum(hp, 0.0)


def _author1_body(aggr, cr, xa, wlr, wra, ba, xa1):
    invr = 1.0 / jnp.maximum(cr[...], 1.0)
    f32 = jnp.float32
    ha = (jnp.dot(aggr[...] * invr, wlr[...], preferred_element_type=f32)
          + jnp.dot(xa[...], wra[...], preferred_element_type=f32) + ba[...])
    xa1[...] = jnp.maximum(ha, 0.0)


def _layer2_body(i, aggc, aggw, cc, cw, xp1, bat, wlc, wlw, wrp, bp,
                 wlin, blin, pooled, cntb, final):
    invc = 1.0 / jnp.maximum(cc[...], 1.0)
    invw = 1.0 / jnp.maximum(cw[...], 1.0)
    f32 = jnp.float32
    hp = (jnp.dot(aggc[...] * invc, wlc[...], preferred_element_type=f32)
          + jnp.dot(aggw[...] * invw, wlw[...], preferred_element_type=f32)
          + jnp.dot(xp1[...], wrp[...], preferred_element_type=f32) + bp[...])
    xp2 = jnp.maximum(hp, 0.0)
    xp2 = jnp.where(xp2 < jnp.float32(1e30), xp2, jnp.float32(0.0))
    bb = bat[0]
    oh = (lax.broadcasted_iota(jnp.int32, (G, RB), 0)
          == jnp.broadcast_to(bb, (G, RB))).astype(f32)

    @pl.when(i == 0)
    def _():
        pooled[...] = jnp.zeros((G, D), f32)
        cntb[...] = jnp.zeros((G, D), f32)

    pooled[...] += jnp.dot(oh, xp2, preferred_element_type=f32)
    cntb[...] += jnp.broadcast_to(jnp.sum(oh, axis=1, keepdims=True), (G, D))

    @pl.when(i == GB - 1)
    def _():
        inv = 1.0 / jnp.maximum(cntb[...], 1.0)
        final[...] = (jnp.dot(pooled[...] * inv, wlin[...],
                              preferred_element_type=f32) + blin[...])


def _l2_with_i(*args):
    _layer2_body(pl.program_id(0), *args)


_row = pl.BlockSpec((RB, D), lambda i: (i, 0))
_col1 = pl.BlockSpec((RB, 1), lambda i: (i, 0))
_wfull = pl.BlockSpec((D, D), lambda i: (0, 0))
_bfull = pl.BlockSpec((1, D), lambda i: (0, 0))

_paper1 = pl.pallas_call(
    _paper1_body,
    grid=(GB,),
    in_specs=[_row, _row, _col1, _col1, _row, _wfull, _wfull, _wfull,
              _bfull],
    out_specs=_row,
    out_shape=jax.ShapeDtypeStruct((NP, D), jnp.float32),
    compiler_params=pltpu.CompilerParams(
        dimension_semantics=("arbitrary",)),
)

_author1 = pl.pallas_call(
    _author1_body,
    grid=(GB,),
    in_specs=[_row, _col1, _row, _wfull, _wfull, _bfull],
    out_specs=_row,
    out_shape=jax.ShapeDtypeStruct((NP, D), jnp.float32),
    compiler_params=pltpu.CompilerParams(
        dimension_semantics=("arbitrary",)),
)

_layer2 = pl.pallas_call(
    _l2_with_i,
    grid=(GB,),
    in_specs=[_row, _row, _col1, _col1, _row,
              pl.BlockSpec((1, 1, RB), lambda i: (i, 0, 0)),
              _wfull, _wfull, _wfull, _bfull,
              pl.BlockSpec((D, C), lambda i: (0, 0)),
              pl.BlockSpec((1, C), lambda i: (0, 0))],
    out_specs=[pl.BlockSpec((G, D), lambda i: (0, 0)),
               pl.BlockSpec((G, D), lambda i: (0, 0)),
               pl.BlockSpec((G, C), lambda i: (0, 0))],
    out_shape=[jax.ShapeDtypeStruct((G, D), jnp.float32),
               jax.ShapeDtypeStruct((G, D), jnp.float32),
               jax.ShapeDtypeStruct((G, C), jnp.float32)],
    compiler_params=pltpu.CompilerParams(
        dimension_semantics=("arbitrary",)),
)


def _pad_edges(ei):
    src = jnp.concatenate([ei[0], jnp.zeros((EP - E,), jnp.int32)])
    dst = jnp.concatenate([ei[1], jnp.full((EP - E,), 1 << 28, jnp.int32)])
    return src.reshape(NS, EPT // 16, 16), dst.reshape(NS, EPT // 16, 16)


def kernel(x_paper, x_author, edge_index_cites, edge_index_writes,
           edge_index_rev, batch, Wl1c, bl1c, Wr1c, Wl1w, bl1w, Wr1w,
           Wl1r, bl1r, Wr1r, Wl2c, bl2c, Wr2c, Wl2w, bl2w, Wr2w,
           Wl2r, bl2r, Wr2r, Wlin, blin):
    xp = x_paper
    xa = x_author
    sc_, dc_ = _pad_edges(edge_index_cites)
    sw_, dw_ = _pad_edges(edge_index_writes)
    sr_, dr_ = _pad_edges(edge_index_rev)
    bat = jnp.concatenate([batch, jnp.full((NP - N,), G, jnp.int32)])
    bat = bat.reshape(GB, 1, RB)

    aggc, aggw, cc, cw = _agg_cw(xp, xa, sc_, dc_, sw_, dw_)
    aggr, cr = _agg_r(xp, sr_, dr_)
    cc = cc.reshape(NP, 1)
    cw = cw.reshape(NP, 1)
    cr = cr.reshape(NP, 1)

    xp1 = _paper1(aggc, aggw, cc, cw, xp, Wl1c, Wl1w, (Wr1c + Wr1w),
                  (bl1c + bl1w).reshape(1, D))
    xa1 = _author1(aggr, cr, xa, Wl1r, Wr1r, bl1r.reshape(1, D))

    (aggc2,) = _agg1(xp1, sc_, dc_)
    (aggw2,) = _agg1(xa1, sw_, dw_)

    _, _, final = _layer2(aggc2, aggw2, cc, cw, xp1, bat,
                          Wl2c, Wl2w, (Wr2c + Wr2w),
                          (bl2c + bl2w).reshape(1, D),
                          Wlin, blin.reshape(1, C))
    return final
